# Initial kernel scaffold; baseline (speedup 1.0000x reference)
import jax
import jax.numpy as jnp
from jax.experimental import pallas as pl


def kernel(x, edge_index, norm, edge_rel_emd, target_rel_emd_new, W1, b1, W2, b2, Wsl, Wline, bline):
    raise NotImplementedError("write your pallas kernel here")


# speedup vs baseline: 1.3051x; 1.3051x over previous
import jax
import jax.numpy as jnp
from jax import lax
from jax.experimental import pallas as pl
from jax.experimental.pallas import tpu as pltpu
from jax.experimental.pallas import tpu_sc as plsc

N_NODES = 10000
N_EDGES = 320000
D = 128
H = 64
NPAD = 10016
NS = 16
NW = 32
EPS = N_EDGES // NS
CH = 40
NCH = EPS // CH
ROWS_PER_SUB = NPAD // NS
DEPS = N_EDGES // NW
DCH = 80
DNCH = DEPS // DCH

_SC_PARAMS = pltpu.CompilerParams(use_tc_tiling_on_sc=False,
                                  needs_layout_passes=False)


def _pre_body(x_ref, w1_ref, b1_ref, wsl_ref, a_ref, b_ref, xsl_ref):
    xb = x_ref[...]
    h = jnp.dot(xb, w1_ref[...], preferred_element_type=jnp.float32) + b1_ref[...]
    a_ref[...] = h[:, :H]
    b_ref[...] = h[:, H:]
    xsl_ref[...] = jnp.dot(xb, wsl_ref[...], preferred_element_type=jnp.float32)


def _pre(x, W1, b1r, Wsl):
    B = 2000
    g = N_NODES // B
    return pl.pallas_call(
        _pre_body,
        grid=(g,),
        in_specs=[
            pl.BlockSpec((B, D), lambda i: (i, 0)),
            pl.BlockSpec((D, D), lambda i: (0, 0)),
            pl.BlockSpec((1, D), lambda i: (0, 0)),
            pl.BlockSpec((D, D), lambda i: (0, 0)),
        ],
        out_specs=[
            pl.BlockSpec((B, H), lambda i: (i, 0)),
            pl.BlockSpec((B, H), lambda i: (i, 0)),
            pl.BlockSpec((B, D), lambda i: (i, 0)),
        ],
        out_shape=[
            jax.ShapeDtypeStruct((N_NODES, H), jnp.float32),
            jax.ShapeDtypeStruct((N_NODES, H), jnp.float32),
            jax.ShapeDtypeStruct((N_NODES, D), jnp.float32),
        ],
    )(x, W1, b1r, Wsl)


def _sc_body(src_hbm, dst_hbm, erel_hbm, xw1a_hbm, xw1b_hbm,
             outE_hbm, outW_hbm, outP_hbm,
             accE, accW, accP,
             srcb, dstb, erelb, w1b, prodb, sem):
    c = lax.axis_index("c")
    s = lax.axis_index("s")

    zeros16 = jnp.zeros((16,), jnp.float32)

    def _zb_zero(t, _):
        erelb[t // 4, pl.ds((t % 4) * 16, 16)] = zeros16
        return _
    lax.fori_loop(0, CH * 4, _zb_zero, ())

    r0 = s * ROWS_PER_SUB
    left = ROWS_PER_SUB
    off = 0
    while left > 0:
        nb = min(CH, left)
        pltpu.sync_copy(erelb.at[pl.ds(0, nb)], accE.at[pl.ds(r0 + off, nb)])
        pltpu.sync_copy(erelb.at[pl.ds(0, nb)], accW.at[pl.ds(r0 + off, nb)])
        pltpu.sync_copy(erelb.at[pl.ds(0, nb)], accP.at[pl.ds(r0 + off, nb)])
        off += nb
        left -= nb
    plsc.subcore_barrier()

    def run(col0, xw1_hbm):
        e_base = s * EPS

        def _chunk_body(k, _):
            e0 = e_base + k * CH
            pltpu.sync_copy(src_hbm.at[pl.ds(e0, CH)], srcb)
            pltpu.sync_copy(dst_hbm.at[pl.ds(e0, CH)], dstb)
            pltpu.async_copy(xw1_hbm.at[srcb], w1b, sem).wait()
            pltpu.sync_copy(erel_hbm.at[pl.ds(e0, CH), pl.ds(col0, H)], erelb)

            def _mul(t, _i):
                i = t // 4
                j = (t % 4) * 16
                prodb[i, pl.ds(j, 16)] = erelb[i, pl.ds(j, 16)] * w1b[i, pl.ds(j, 16)]
                return _i
            lax.fori_loop(0, CH * 4, _mul, ())

            pltpu.sync_copy(erelb, accE.at[dstb], add=True)
            pltpu.sync_copy(w1b, accW.at[dstb], add=True)
            pltpu.sync_copy(prodb, accP.at[dstb], add=True)
            return _
        lax.fori_loop(0, NCH, _chunk_body, ())

    pl.when(c == 0)(lambda: run(0, xw1a_hbm))
    pl.when(c == 1)(lambda: run(H, xw1b_hbm))

    plsc.subcore_barrier()

    out_r0 = c * NPAD + r0
    pltpu.sync_copy(accE.at[pl.ds(r0, ROWS_PER_SUB)], outE_hbm.at[pl.ds(out_r0, ROWS_PER_SUB)])
    pltpu.sync_copy(accW.at[pl.ds(r0, ROWS_PER_SUB)], outW_hbm.at[pl.ds(out_r0, ROWS_PER_SUB)])
    pltpu.sync_copy(accP.at[pl.ds(r0, ROWS_PER_SUB)], outP_hbm.at[pl.ds(out_r0, ROWS_PER_SUB)])


def _sc(src, dst, erel, xw1a, xw1b):
    mesh = plsc.VectorSubcoreMesh(core_axis_name="c", subcore_axis_name="s")
    f = pl.kernel(
        _sc_body,
        out_type=[
            jax.ShapeDtypeStruct((2 * NPAD, H), jnp.float32),
            jax.ShapeDtypeStruct((2 * NPAD, H), jnp.float32),
            jax.ShapeDtypeStruct((2 * NPAD, H), jnp.float32),
        ],
        mesh=mesh,
        compiler_params=_SC_PARAMS,
        scratch_types=[
            pltpu.VMEM_SHARED((NPAD, H), jnp.float32),
            pltpu.VMEM_SHARED((NPAD, H), jnp.float32),
            pltpu.VMEM_SHARED((NPAD, H), jnp.float32),
            pltpu.VMEM((CH,), jnp.int32),
            pltpu.VMEM((CH,), jnp.int32),
            pltpu.VMEM((CH, H), jnp.float32),
            pltpu.VMEM((CH, H), jnp.float32),
            pltpu.VMEM((CH, H), jnp.float32),
            pltpu.SemaphoreType.DMA,
        ],
    )
    return f(src, dst, erel, xw1a, xw1b)


def _deg_body(dst_hbm, dego_hbm, degl, dstb):
    c = lax.axis_index("c")
    s = lax.axis_index("s")
    w = s * 2 + c

    zeros16 = jnp.zeros((16,), jnp.float32)
    ones16 = jnp.ones((16,), jnp.float32)

    def _deg_zero(t, _):
        degl[pl.ds(t * 16, 16)] = zeros16
        return _
    lax.fori_loop(0, NPAD // 16, _deg_zero, ())

    e_base = w * DEPS

    def _chunk_body(k, _):
        pltpu.sync_copy(dst_hbm.at[pl.ds(e_base + k * DCH, DCH)], dstb)

        def _acc(j, _i):
            idx16 = dstb[pl.ds(j * 16, 16)]
            plsc.addupdate_scatter(degl, [idx16], ones16)
            return _i
        lax.fori_loop(0, DCH // 16, _acc, ())
        return _
    lax.fori_loop(0, DNCH, _chunk_body, ())

    pltpu.sync_copy(degl, dego_hbm.at[pl.ds(w * NPAD, NPAD)])


def _scdeg(dst):
    mesh = plsc.VectorSubcoreMesh(core_axis_name="c", subcore_axis_name="s")
    f = pl.kernel(
        _deg_body,
        out_type=jax.ShapeDtypeStruct((NW * NPAD,), jnp.float32),
        mesh=mesh,
        compiler_params=_SC_PARAMS,
        scratch_types=[
            pltpu.VMEM((NPAD,), jnp.float32),
            pltpu.VMEM((DCH,), jnp.int32),
        ],
    )
    return f(dst)


def _post_body(e3_ref, w3_ref, p3_ref, dgp_ref, xsl_ref,
               w2_ref, b2_ref, wline_ref, bline_ref, out_ref):
    Se = jnp.concatenate([e3_ref[0], e3_ref[1]], axis=1)
    Sw = jnp.concatenate([w3_ref[0], w3_ref[1]], axis=1)
    Sp = jnp.concatenate([p3_ref[0], p3_ref[1]], axis=1)
    deg = jnp.sum(dgp_ref[...], axis=1)
    w2 = w2_ref[...]
    W2a = w2[0:D]
    W2b = w2[D:2 * D]
    W2c = w2[2 * D:3 * D]
    nei = (jnp.dot(Se, W2a + W2b, preferred_element_type=jnp.float32)
           + jnp.dot(Sw, W2a - W2b, preferred_element_type=jnp.float32)
           + jnp.dot(Sp, W2c, preferred_element_type=jnp.float32)
           + deg[:, None] * b2_ref[...])
    has_in = (deg > 0).astype(jnp.float32)[:, None]
    node = nei + xsl_ref[...] * has_in
    out = jnp.dot(node, wline_ref[...], preferred_element_type=jnp.float32) + bline_ref[...]
    out_ref[...] = jnp.maximum(out, 0.0)


def _post(e3, w3, p3, dgp, xsl, W2, b2r, Wline, bliner):
    B = 2000
    g = N_NODES // B
    return pl.pallas_call(
        _post_body,
        grid=(g,),
        in_specs=[
            pl.BlockSpec((2, B, H), lambda i: (0, i, 0)),
            pl.BlockSpec((2, B, H), lambda i: (0, i, 0)),
            pl.BlockSpec((2, B, H), lambda i: (0, i, 0)),
            pl.BlockSpec((B, NW), lambda i: (i, 0)),
            pl.BlockSpec((B, D), lambda i: (i, 0)),
            pl.BlockSpec((3 * D, D), lambda i: (0, 0)),
            pl.BlockSpec((1, D), lambda i: (0, 0)),
            pl.BlockSpec((D, D), lambda i: (0, 0)),
            pl.BlockSpec((1, D), lambda i: (0, 0)),
        ],
        out_specs=pl.BlockSpec((B, D), lambda i: (i, 0)),
        out_shape=jax.ShapeDtypeStruct((N_NODES, D), jnp.float32),
    )(e3, w3, p3, dgp, xsl, W2, b2r, Wline, bliner)


def kernel(x, edge_index, norm, edge_rel_emd, target_rel_emd_new,
           W1, b1, W2, b2, Wsl, Wline, bline):
    del norm, target_rel_emd_new
    src = edge_index[0]
    dst = edge_index[1]
    xw1a, xw1b, xsl = _pre(x, W1, b1.reshape(1, D), Wsl)
    outE, outW, outP = _sc(src, dst, edge_rel_emd, xw1a, xw1b)
    degp = _scdeg(dst)
    e3 = outE.reshape(2, NPAD, H)
    w3 = outW.reshape(2, NPAD, H)
    p3 = outP.reshape(2, NPAD, H)
    dgp = degp.reshape(NW, NPAD).T
    return _post(e3, w3, p3, dgp, xsl, W2, b2.reshape(1, D),
                 Wline, bline.reshape(1, D))


# speedup vs baseline: 1.4930x; 1.1440x over previous
import jax
import jax.numpy as jnp
from jax import lax
from jax.experimental import pallas as pl
from jax.experimental.pallas import tpu as pltpu
from jax.experimental.pallas import tpu_sc as plsc

N_NODES = 10000
N_EDGES = 320000
D = 128
H = 64
NPAD = 10016
NS = 16
NW = 32
EPS = N_EDGES // NS
CH = 40
NCH = EPS // CH
ROWS_PER_SUB = NPAD // NS
DEPS = N_EDGES // NW
DCH = 80
DNCH = DEPS // DCH

_SC_PARAMS = pltpu.CompilerParams(use_tc_tiling_on_sc=False,
                                  needs_layout_passes=False)


def _pre_body(x_ref, w1_ref, b1_ref, wsl_ref, a_ref, b_ref, xsl_ref):
    xb = x_ref[...]
    h = jnp.dot(xb, w1_ref[...], preferred_element_type=jnp.float32) + b1_ref[...]
    a_ref[...] = h[:, :H]
    b_ref[...] = h[:, H:]
    xsl_ref[...] = jnp.dot(xb, wsl_ref[...], preferred_element_type=jnp.float32)


def _pre(x, W1, b1r, Wsl):
    B = 2000
    g = N_NODES // B
    return pl.pallas_call(
        _pre_body,
        grid=(g,),
        in_specs=[
            pl.BlockSpec((B, D), lambda i: (i, 0)),
            pl.BlockSpec((D, D), lambda i: (0, 0)),
            pl.BlockSpec((1, D), lambda i: (0, 0)),
            pl.BlockSpec((D, D), lambda i: (0, 0)),
        ],
        out_specs=[
            pl.BlockSpec((B, H), lambda i: (i, 0)),
            pl.BlockSpec((B, H), lambda i: (i, 0)),
            pl.BlockSpec((B, D), lambda i: (i, 0)),
        ],
        out_shape=[
            jax.ShapeDtypeStruct((N_NODES, H), jnp.float32),
            jax.ShapeDtypeStruct((N_NODES, H), jnp.float32),
            jax.ShapeDtypeStruct((N_NODES, D), jnp.float32),
        ],
    )(x, W1, b1r, Wsl)


def _sc_body(src_hbm, dst_hbm, erel_hbm, xw1a_hbm, xw1b_hbm,
             outE_hbm, outW_hbm, outP_hbm,
             accE, accW, accP,
             srcb, dstb, erelb, w1b, prodb, sem):
    c = lax.axis_index("c")
    s = lax.axis_index("s")

    zeros16 = jnp.zeros((16,), jnp.float32)

    def _zb_zero(i, _):
        for j in range(4):
            erelb[i, pl.ds(j * 16, 16)] = zeros16
        return _
    lax.fori_loop(0, CH, _zb_zero, ())

    r0 = s * ROWS_PER_SUB
    left = ROWS_PER_SUB
    off = 0
    while left > 0:
        nb = min(CH, left)
        pltpu.sync_copy(erelb.at[pl.ds(0, nb)], accE.at[pl.ds(r0 + off, nb)])
        pltpu.sync_copy(erelb.at[pl.ds(0, nb)], accW.at[pl.ds(r0 + off, nb)])
        pltpu.sync_copy(erelb.at[pl.ds(0, nb)], accP.at[pl.ds(r0 + off, nb)])
        off += nb
        left -= nb
    plsc.subcore_barrier()

    def run(col0, xw1_hbm):
        e_base = s * EPS

        def _chunk_body(k, _):
            e0 = e_base + k * CH
            pltpu.sync_copy(src_hbm.at[pl.ds(e0, CH)], srcb)
            pltpu.sync_copy(dst_hbm.at[pl.ds(e0, CH)], dstb)
            pltpu.async_copy(xw1_hbm.at[srcb], w1b, sem).wait()
            pltpu.sync_copy(erel_hbm.at[pl.ds(e0, CH), pl.ds(col0, H)], erelb)

            def _mul(i, _i):
                for j in range(0, 64, 16):
                    prodb[i, pl.ds(j, 16)] = erelb[i, pl.ds(j, 16)] * w1b[i, pl.ds(j, 16)]
                return _i
            lax.fori_loop(0, CH, _mul, ())

            pltpu.sync_copy(erelb, accE.at[dstb], add=True)
            pltpu.sync_copy(w1b, accW.at[dstb], add=True)
            pltpu.sync_copy(prodb, accP.at[dstb], add=True)
            return _
        lax.fori_loop(0, NCH, _chunk_body, ())

    pl.when(c == 0)(lambda: run(0, xw1a_hbm))
    pl.when(c == 1)(lambda: run(H, xw1b_hbm))

    plsc.subcore_barrier()

    out_r0 = c * NPAD + r0
    pltpu.sync_copy(accE.at[pl.ds(r0, ROWS_PER_SUB)], outE_hbm.at[pl.ds(out_r0, ROWS_PER_SUB)])
    pltpu.sync_copy(accW.at[pl.ds(r0, ROWS_PER_SUB)], outW_hbm.at[pl.ds(out_r0, ROWS_PER_SUB)])
    pltpu.sync_copy(accP.at[pl.ds(r0, ROWS_PER_SUB)], outP_hbm.at[pl.ds(out_r0, ROWS_PER_SUB)])


def _sc(src, dst, erel, xw1a, xw1b):
    mesh = plsc.VectorSubcoreMesh(core_axis_name="c", subcore_axis_name="s")
    f = pl.kernel(
        _sc_body,
        out_type=[
            jax.ShapeDtypeStruct((2 * NPAD, H), jnp.float32),
            jax.ShapeDtypeStruct((2 * NPAD, H), jnp.float32),
            jax.ShapeDtypeStruct((2 * NPAD, H), jnp.float32),
        ],
        mesh=mesh,
        compiler_params=_SC_PARAMS,
        scratch_types=[
            pltpu.VMEM_SHARED((NPAD, H), jnp.float32),
            pltpu.VMEM_SHARED((NPAD, H), jnp.float32),
            pltpu.VMEM_SHARED((NPAD, H), jnp.float32),
            pltpu.VMEM((CH,), jnp.int32),
            pltpu.VMEM((CH,), jnp.int32),
            pltpu.VMEM((CH, H), jnp.float32),
            pltpu.VMEM((CH, H), jnp.float32),
            pltpu.VMEM((CH, H), jnp.float32),
            pltpu.SemaphoreType.DMA,
        ],
    )
    return f(src, dst, erel, xw1a, xw1b)


def _deg_body(dst_hbm, dego_hbm, degl, dstb):
    c = lax.axis_index("c")
    s = lax.axis_index("s")
    w = s * 2 + c

    zeros16 = jnp.zeros((16,), jnp.float32)
    ones16 = jnp.ones((16,), jnp.float32)

    def _deg_zero(t, _):
        degl[pl.ds(t * 16, 16)] = zeros16
        return _
    lax.fori_loop(0, NPAD // 16, _deg_zero, ())

    e_base = w * DEPS

    def _chunk_body(k, _):
        pltpu.sync_copy(dst_hbm.at[pl.ds(e_base + k * DCH, DCH)], dstb)

        def _acc(j, _i):
            idx16 = dstb[pl.ds(j * 16, 16)]
            plsc.addupdate_scatter(degl, [idx16], ones16)
            return _i
        lax.fori_loop(0, DCH // 16, _acc, ())
        return _
    lax.fori_loop(0, DNCH, _chunk_body, ())

    pltpu.sync_copy(degl, dego_hbm.at[pl.ds(w * NPAD, NPAD)])


def _scdeg(dst):
    mesh = plsc.VectorSubcoreMesh(core_axis_name="c", subcore_axis_name="s")
    f = pl.kernel(
        _deg_body,
        out_type=jax.ShapeDtypeStruct((NW * NPAD,), jnp.float32),
        mesh=mesh,
        compiler_params=_SC_PARAMS,
        scratch_types=[
            pltpu.VMEM((NPAD,), jnp.float32),
            pltpu.VMEM((DCH,), jnp.int32),
        ],
    )
    return f(dst)


def _post_body(e3_ref, w3_ref, p3_ref, dgp_ref, xsl_ref,
               w2_ref, b2_ref, wline_ref, bline_ref, out_ref):
    Se = jnp.concatenate([e3_ref[0], e3_ref[1]], axis=1)
    Sw = jnp.concatenate([w3_ref[0], w3_ref[1]], axis=1)
    Sp = jnp.concatenate([p3_ref[0], p3_ref[1]], axis=1)
    deg = jnp.sum(dgp_ref[...], axis=1)
    w2 = w2_ref[...]
    W2a = w2[0:D]
    W2b = w2[D:2 * D]
    W2c = w2[2 * D:3 * D]
    nei = (jnp.dot(Se, W2a + W2b, preferred_element_type=jnp.float32)
           + jnp.dot(Sw, W2a - W2b, preferred_element_type=jnp.float32)
           + jnp.dot(Sp, W2c, preferred_element_type=jnp.float32)
           + deg[:, None] * b2_ref[...])
    has_in = (deg > 0).astype(jnp.float32)[:, None]
    node = nei + xsl_ref[...] * has_in
    out = jnp.dot(node, wline_ref[...], preferred_element_type=jnp.float32) + bline_ref[...]
    out_ref[...] = jnp.maximum(out, 0.0)


def _post(e3, w3, p3, dgp, xsl, W2, b2r, Wline, bliner):
    B = 2000
    g = N_NODES // B
    return pl.pallas_call(
        _post_body,
        grid=(g,),
        in_specs=[
            pl.BlockSpec((2, B, H), lambda i: (0, i, 0)),
            pl.BlockSpec((2, B, H), lambda i: (0, i, 0)),
            pl.BlockSpec((2, B, H), lambda i: (0, i, 0)),
            pl.BlockSpec((B, NW), lambda i: (i, 0)),
            pl.BlockSpec((B, D), lambda i: (i, 0)),
            pl.BlockSpec((3 * D, D), lambda i: (0, 0)),
            pl.BlockSpec((1, D), lambda i: (0, 0)),
            pl.BlockSpec((D, D), lambda i: (0, 0)),
            pl.BlockSpec((1, D), lambda i: (0, 0)),
        ],
        out_specs=pl.BlockSpec((B, D), lambda i: (i, 0)),
        out_shape=jax.ShapeDtypeStruct((N_NODES, D), jnp.float32),
    )(e3, w3, p3, dgp, xsl, W2, b2r, Wline, bliner)


def kernel(x, edge_index, norm, edge_rel_emd, target_rel_emd_new,
           W1, b1, W2, b2, Wsl, Wline, bline):
    del norm, target_rel_emd_new
    src = edge_index[0]
    dst = edge_index[1]
    xw1a, xw1b, xsl = _pre(x, W1, b1.reshape(1, D), Wsl)
    outE, outW, outP = _sc(src, dst, edge_rel_emd, xw1a, xw1b)
    degp = _scdeg(dst)
    e3 = outE.reshape(2, NPAD, H)
    w3 = outW.reshape(2, NPAD, H)
    p3 = outP.reshape(2, NPAD, H)
    dgp = degp.reshape(NW, NPAD).T
    return _post(e3, w3, p3, dgp, xsl, W2, b2.reshape(1, D),
                 Wline, bline.reshape(1, D))


# speedup vs baseline: 2.2562x; 1.5111x over previous
import jax
import jax.numpy as jnp
from jax import lax
from jax.experimental import pallas as pl
from jax.experimental.pallas import tpu as pltpu
from jax.experimental.pallas import tpu_sc as plsc

N_NODES = 10000
N_EDGES = 320000
D = 128
H = 64
NPAD = 10016
NS = 16
NW = 32
EPS = N_EDGES // NS
CH = 40
NCH = EPS // CH
ROWS_PER_SUB = NPAD // NS
DEPS = N_EDGES // NW
DCH = 80
DNCH = DEPS // DCH

_SC_PARAMS = pltpu.CompilerParams(use_tc_tiling_on_sc=False,
                                  needs_layout_passes=False)


def _pre_body(x_ref, w1_ref, b1_ref, wsl_ref, a_ref, b_ref, xsl_ref):
    xb = x_ref[...]
    h = jnp.dot(xb, w1_ref[...], preferred_element_type=jnp.float32) + b1_ref[...]
    a_ref[...] = h[:, :H]
    b_ref[...] = h[:, H:]
    xsl_ref[...] = jnp.dot(xb, wsl_ref[...], preferred_element_type=jnp.float32)


def _pre(x, W1, b1r, Wsl):
    B = 2000
    g = N_NODES // B
    return pl.pallas_call(
        _pre_body,
        grid=(g,),
        in_specs=[
            pl.BlockSpec((B, D), lambda i: (i, 0)),
            pl.BlockSpec((D, D), lambda i: (0, 0)),
            pl.BlockSpec((1, D), lambda i: (0, 0)),
            pl.BlockSpec((D, D), lambda i: (0, 0)),
        ],
        out_specs=[
            pl.BlockSpec((B, H), lambda i: (i, 0)),
            pl.BlockSpec((B, H), lambda i: (i, 0)),
            pl.BlockSpec((B, D), lambda i: (i, 0)),
        ],
        out_shape=[
            jax.ShapeDtypeStruct((N_NODES, H), jnp.float32),
            jax.ShapeDtypeStruct((N_NODES, H), jnp.float32),
            jax.ShapeDtypeStruct((N_NODES, D), jnp.float32),
        ],
    )(x, W1, b1r, Wsl)


def _sc_body(src_hbm, dst_hbm, erel_hbm, xw1a_hbm, xw1b_hbm,
             outE_hbm, outW_hbm, outP_hbm,
             accE, accW, accP,
             srcb, dstb, erelb, w1b, prodb,
             ses, sed, see, seg):
    c = lax.axis_index("c")
    s = lax.axis_index("s")

    zeros16 = jnp.zeros((16,), jnp.float32)

    def _zb_zero(i, _):
        for j in range(4):
            erelb[i, pl.ds(j * 16, 16)] = zeros16
        return _
    lax.fori_loop(0, CH, _zb_zero, ())

    r0 = s * ROWS_PER_SUB
    left = ROWS_PER_SUB
    off = 0
    while left > 0:
        nb = min(CH, left)
        pltpu.sync_copy(erelb.at[pl.ds(0, nb)], accE.at[pl.ds(r0 + off, nb)])
        pltpu.sync_copy(erelb.at[pl.ds(0, nb)], accW.at[pl.ds(r0 + off, nb)])
        pltpu.sync_copy(erelb.at[pl.ds(0, nb)], accP.at[pl.ds(r0 + off, nb)])
        off += nb
        left -= nb
    plsc.subcore_barrier()

    def run(col0, xw1_hbm):
        e_base = s * EPS

        def _chunk_body(k, _):
            e0 = e_base + k * CH
            cs = pltpu.async_copy(src_hbm.at[pl.ds(e0, CH)], srcb, ses)
            cd = pltpu.async_copy(dst_hbm.at[pl.ds(e0, CH)], dstb, sed)
            ce = pltpu.async_copy(erel_hbm.at[pl.ds(e0, CH), pl.ds(col0, H)],
                                  erelb, see)
            cs.wait()
            gat = pltpu.async_copy(xw1_hbm.at[srcb], w1b, seg)
            ce.wait()
            gat.wait()

            def _mul(i, _i):
                for j in range(0, 64, 16):
                    prodb[i, pl.ds(j, 16)] = (erelb[i, pl.ds(j, 16)]
                                              * w1b[i, pl.ds(j, 16)])
                return _i
            lax.fori_loop(0, CH, _mul, ())

            cd.wait()
            pltpu.sync_copy(erelb, accE.at[dstb], add=True)
            pltpu.sync_copy(w1b, accW.at[dstb], add=True)
            pltpu.sync_copy(prodb, accP.at[dstb], add=True)
            return _
        lax.fori_loop(0, NCH, _chunk_body, ())

    pl.when(c == 0)(lambda: run(0, xw1a_hbm))
    pl.when(c == 1)(lambda: run(H, xw1b_hbm))

    plsc.subcore_barrier()

    out_r0 = c * NPAD + r0
    pltpu.sync_copy(accE.at[pl.ds(r0, ROWS_PER_SUB)], outE_hbm.at[pl.ds(out_r0, ROWS_PER_SUB)])
    pltpu.sync_copy(accW.at[pl.ds(r0, ROWS_PER_SUB)], outW_hbm.at[pl.ds(out_r0, ROWS_PER_SUB)])
    pltpu.sync_copy(accP.at[pl.ds(r0, ROWS_PER_SUB)], outP_hbm.at[pl.ds(out_r0, ROWS_PER_SUB)])


def _sc(src, dst, erel, xw1a, xw1b):
    mesh = plsc.VectorSubcoreMesh(core_axis_name="c", subcore_axis_name="s")
    f = pl.kernel(
        _sc_body,
        out_type=[
            jax.ShapeDtypeStruct((2 * NPAD, H), jnp.float32),
            jax.ShapeDtypeStruct((2 * NPAD, H), jnp.float32),
            jax.ShapeDtypeStruct((2 * NPAD, H), jnp.float32),
        ],
        mesh=mesh,
        compiler_params=_SC_PARAMS,
        scratch_types=[
            pltpu.VMEM_SHARED((NPAD, H), jnp.float32),
            pltpu.VMEM_SHARED((NPAD, H), jnp.float32),
            pltpu.VMEM_SHARED((NPAD, H), jnp.float32),
            pltpu.VMEM((CH,), jnp.int32),
            pltpu.VMEM((CH,), jnp.int32),
            pltpu.VMEM((CH, H), jnp.float32),
            pltpu.VMEM((CH, H), jnp.float32),
            pltpu.VMEM((CH, H), jnp.float32),
            pltpu.SemaphoreType.DMA,
            pltpu.SemaphoreType.DMA,
            pltpu.SemaphoreType.DMA,
            pltpu.SemaphoreType.DMA,
        ],
    )
    return f(src, dst, erel, xw1a, xw1b)


def _deg_body(dst_hbm, dego_hbm, degl, dstb):
    c = lax.axis_index("c")
    s = lax.axis_index("s")
    w = s * 2 + c

    zeros16 = jnp.zeros((16,), jnp.float32)
    ones16 = jnp.ones((16,), jnp.float32)

    def _deg_zero(t, _):
        degl[pl.ds(t * 16, 16)] = zeros16
        return _
    lax.fori_loop(0, NPAD // 16, _deg_zero, ())

    e_base = w * DEPS

    def _chunk_body(k, _):
        pltpu.sync_copy(dst_hbm.at[pl.ds(e_base + k * DCH, DCH)], dstb)

        def _acc(j, _i):
            idx16 = dstb[pl.ds(j * 16, 16)]
            plsc.addupdate_scatter(degl, [idx16], ones16)
            return _i
        lax.fori_loop(0, DCH // 16, _acc, ())
        return _
    lax.fori_loop(0, DNCH, _chunk_body, ())

    pltpu.sync_copy(degl, dego_hbm.at[pl.ds(w * NPAD, NPAD)])


def _scdeg(dst):
    mesh = plsc.VectorSubcoreMesh(core_axis_name="c", subcore_axis_name="s")
    f = pl.kernel(
        _deg_body,
        out_type=jax.ShapeDtypeStruct((NW * NPAD,), jnp.float32),
        mesh=mesh,
        compiler_params=_SC_PARAMS,
        scratch_types=[
            pltpu.VMEM((NPAD,), jnp.float32),
            pltpu.VMEM((DCH,), jnp.int32),
        ],
    )
    return f(dst)


def _post_body(e3_ref, w3_ref, p3_ref, dgp_ref, xsl_ref,
               w2_ref, b2_ref, wline_ref, bline_ref, out_ref):
    Se = jnp.concatenate([e3_ref[0], e3_ref[1]], axis=1)
    Sw = jnp.concatenate([w3_ref[0], w3_ref[1]], axis=1)
    Sp = jnp.concatenate([p3_ref[0], p3_ref[1]], axis=1)
    deg = jnp.sum(dgp_ref[...], axis=1)
    w2 = w2_ref[...]
    W2a = w2[0:D]
    W2b = w2[D:2 * D]
    W2c = w2[2 * D:3 * D]
    nei = (jnp.dot(Se, W2a + W2b, preferred_element_type=jnp.float32)
           + jnp.dot(Sw, W2a - W2b, preferred_element_type=jnp.float32)
           + jnp.dot(Sp, W2c, preferred_element_type=jnp.float32)
           + deg[:, None] * b2_ref[...])
    has_in = (deg > 0).astype(jnp.float32)[:, None]
    node = nei + xsl_ref[...] * has_in
    out = jnp.dot(node, wline_ref[...], preferred_element_type=jnp.float32) + bline_ref[...]
    out_ref[...] = jnp.maximum(out, 0.0)


def _post(e3, w3, p3, dgp, xsl, W2, b2r, Wline, bliner):
    B = 2000
    g = N_NODES // B
    return pl.pallas_call(
        _post_body,
        grid=(g,),
        in_specs=[
            pl.BlockSpec((2, B, H), lambda i: (0, i, 0)),
            pl.BlockSpec((2, B, H), lambda i: (0, i, 0)),
            pl.BlockSpec((2, B, H), lambda i: (0, i, 0)),
            pl.BlockSpec((B, NW), lambda i: (i, 0)),
            pl.BlockSpec((B, D), lambda i: (i, 0)),
            pl.BlockSpec((3 * D, D), lambda i: (0, 0)),
            pl.BlockSpec((1, D), lambda i: (0, 0)),
            pl.BlockSpec((D, D), lambda i: (0, 0)),
            pl.BlockSpec((1, D), lambda i: (0, 0)),
        ],
        out_specs=pl.BlockSpec((B, D), lambda i: (i, 0)),
        out_shape=jax.ShapeDtypeStruct((N_NODES, D), jnp.float32),
    )(e3, w3, p3, dgp, xsl, W2, b2r, Wline, bliner)


def kernel(x, edge_index, norm, edge_rel_emd, target_rel_emd_new,
           W1, b1, W2, b2, Wsl, Wline, bline):
    del norm, target_rel_emd_new
    src = edge_index[0]
    dst = edge_index[1]
    xw1a, xw1b, xsl = _pre(x, W1, b1.reshape(1, D), Wsl)
    outE, outW, outP = _sc(src, dst, edge_rel_emd, xw1a, xw1b)
    degp = _scdeg(dst)
    e3 = outE.reshape(2, NPAD, H)
    w3 = outW.reshape(2, NPAD, H)
    p3 = outP.reshape(2, NPAD, H)
    dgp = degp.reshape(NW, NPAD).T
    return _post(e3, w3, p3, dgp, xsl, W2, b2.reshape(1, D),
                 Wline, bline.reshape(1, D))


# speedup vs baseline: 3.5683x; 1.5816x over previous
import jax
import jax.numpy as jnp
from jax import lax
from jax.experimental import pallas as pl
from jax.experimental.pallas import tpu as pltpu
from jax.experimental.pallas import tpu_sc as plsc

N_NODES = 10000
N_EDGES = 320000
D = 128
W = 32
NG = 4
NPAD = 10016
NS = 16
NW = 32
EPS = N_EDGES // NS
CH = 80
NCH = EPS // CH
NPAIRS = (NCH - 2) // 2
ROWS_PER_SUB = NPAD // NS
DEPS = N_EDGES // NW
DCH = 80
DNCH = DEPS // DCH

_SC_PARAMS = pltpu.CompilerParams(use_tc_tiling_on_sc=False,
                                  needs_layout_passes=False)


def _pre_body(x_ref, w1_ref, b1_ref, wsl_ref,
              g0_ref, g1_ref, g2_ref, g3_ref, xsl_ref):
    xb = x_ref[...]
    h = jnp.dot(xb, w1_ref[...], preferred_element_type=jnp.float32) + b1_ref[...]
    g0_ref[...] = h[:, 0 * W:1 * W]
    g1_ref[...] = h[:, 1 * W:2 * W]
    g2_ref[...] = h[:, 2 * W:3 * W]
    g3_ref[...] = h[:, 3 * W:4 * W]
    xsl_ref[...] = jnp.dot(xb, wsl_ref[...], preferred_element_type=jnp.float32)


def _pre(x, W1, b1r, Wsl):
    B = 2000
    g = N_NODES // B
    return pl.pallas_call(
        _pre_body,
        grid=(g,),
        in_specs=[
            pl.BlockSpec((B, D), lambda i: (i, 0)),
            pl.BlockSpec((D, D), lambda i: (0, 0)),
            pl.BlockSpec((1, D), lambda i: (0, 0)),
            pl.BlockSpec((D, D), lambda i: (0, 0)),
        ],
        out_specs=[
            pl.BlockSpec((B, W), lambda i: (i, 0)),
            pl.BlockSpec((B, W), lambda i: (i, 0)),
            pl.BlockSpec((B, W), lambda i: (i, 0)),
            pl.BlockSpec((B, W), lambda i: (i, 0)),
            pl.BlockSpec((B, D), lambda i: (i, 0)),
        ],
        out_shape=[
            jax.ShapeDtypeStruct((N_NODES, W), jnp.float32),
            jax.ShapeDtypeStruct((N_NODES, W), jnp.float32),
            jax.ShapeDtypeStruct((N_NODES, W), jnp.float32),
            jax.ShapeDtypeStruct((N_NODES, W), jnp.float32),
            jax.ShapeDtypeStruct((N_NODES, D), jnp.float32),
        ],
    )(x, W1, b1r, Wsl)


def _sc_body(src_hbm, dst_hbm, erel_hbm, xg0_hbm, xg1_hbm, xg2_hbm, xg3_hbm,
             outE_hbm, outW_hbm, outP_hbm,
             accE, accW, accP,
             srcb0, srcb1, dstb0, dstb1, erelb0, erelb1,
             w1b0, w1b1, prodb0, prodb1,
             ses0, ses1, sed0, sed1, see0, see1, seg0, seg1):
    c = lax.axis_index("c")
    s = lax.axis_index("s")
    srcb = [srcb0, srcb1]
    dstb = [dstb0, dstb1]
    erelb = [erelb0, erelb1]
    w1b = [w1b0, w1b1]
    prodb = [prodb0, prodb1]
    ses = [ses0, ses1]
    sed = [sed0, sed1]
    see = [see0, see1]
    seg = [seg0, seg1]

    zeros16 = jnp.zeros((16,), jnp.float32)
    r0 = s * ROWS_PER_SUB
    e_base = s * EPS

    def zero_strip():
        def _zb_zero(i, _):
            erelb0[i, pl.ds(0, 16)] = zeros16
            erelb0[i, pl.ds(16, 16)] = zeros16
            return _
        lax.fori_loop(0, CH, _zb_zero, ())
        left = ROWS_PER_SUB
        off = 0
        while left > 0:
            nb = min(CH, left)
            pltpu.sync_copy(erelb0.at[pl.ds(0, nb)], accE.at[pl.ds(r0 + off, nb)])
            pltpu.sync_copy(erelb0.at[pl.ds(0, nb)], accW.at[pl.ds(r0 + off, nb)])
            pltpu.sync_copy(erelb0.at[pl.ds(0, nb)], accP.at[pl.ds(r0 + off, nb)])
            off += nb
            left -= nb

    def phase(col0, xw1_hbm, out_base):
        zero_strip()
        plsc.subcore_barrier()

        def fire_inputs(k, b):
            e0 = e_base + k * CH
            pltpu.async_copy(src_hbm.at[pl.ds(e0, CH)], srcb[b], ses[b])
            pltpu.async_copy(dst_hbm.at[pl.ds(e0, CH)], dstb[b], sed[b])
            pltpu.async_copy(erel_hbm.at[pl.ds(e0, CH), pl.ds(col0, W)],
                             erelb[b], see[b])

        def fire_gather(b):
            pltpu.async_copy(xw1_hbm.at[srcb[b]], w1b[b], seg[b])

        def chunk_step(k, b, prefetch, gather_next):
            nb = 1 - b
            e0 = e_base + k * CH
            pltpu.make_async_copy(xw1_hbm.at[srcb[b]], w1b[b], seg[b]).wait()
            pltpu.make_async_copy(erel_hbm.at[pl.ds(e0, CH), pl.ds(col0, W)],
                                  erelb[b], see[b]).wait()

            def _mul(i, _i):
                for r in range(2):
                    for j in range(0, W, 16):
                        prodb[b][2 * i + r, pl.ds(j, 16)] = (
                            erelb[b][2 * i + r, pl.ds(j, 16)]
                            * w1b[b][2 * i + r, pl.ds(j, 16)])
                return _i
            lax.fori_loop(0, CH // 2, _mul, ())

            if gather_next:
                e1 = e_base + (k + 1) * CH
                pltpu.make_async_copy(src_hbm.at[pl.ds(e1, CH)], srcb[nb],
                                      ses[nb]).wait()
                fire_gather(nb)

            pltpu.make_async_copy(dst_hbm.at[pl.ds(e0, CH)], dstb[b],
                                  sed[b]).wait()
            pltpu.sync_copy(erelb[b], accE.at[dstb[b]], add=True)
            pltpu.sync_copy(w1b[b], accW.at[dstb[b]], add=True)
            pltpu.sync_copy(prodb[b], accP.at[dstb[b]], add=True)
            if prefetch:
                fire_inputs(k + 2, b)

        fire_inputs(0, 0)
        fire_inputs(1, 1)
        pltpu.make_async_copy(src_hbm.at[pl.ds(e_base, CH)], srcb[0],
                              ses[0]).wait()
        fire_gather(0)

        def _pair_body(g, _):
            chunk_step(2 * g, 0, True, True)
            chunk_step(2 * g + 1, 1, True, True)
            return _
        lax.fori_loop(0, NPAIRS, _pair_body, ())
        chunk_step(NCH - 2, 0, False, True)
        chunk_step(NCH - 1, 1, False, False)

        plsc.subcore_barrier()
        out_r0 = out_base + r0
        pltpu.sync_copy(accE.at[pl.ds(r0, ROWS_PER_SUB)],
                        outE_hbm.at[pl.ds(out_r0, ROWS_PER_SUB)])
        pltpu.sync_copy(accW.at[pl.ds(r0, ROWS_PER_SUB)],
                        outW_hbm.at[pl.ds(out_r0, ROWS_PER_SUB)])
        pltpu.sync_copy(accP.at[pl.ds(r0, ROWS_PER_SUB)],
                        outP_hbm.at[pl.ds(out_r0, ROWS_PER_SUB)])

    def run_core(xw1s, gbase):
        for p in range(2):
            g = gbase + p
            phase(g * W, xw1s[p], g * NPAD)

    pl.when(c == 0)(lambda: run_core([xg0_hbm, xg1_hbm], 0))
    pl.when(c == 1)(lambda: run_core([xg2_hbm, xg3_hbm], 2))


def _sc(src, dst, erel, xg0, xg1, xg2, xg3):
    mesh = plsc.VectorSubcoreMesh(core_axis_name="c", subcore_axis_name="s")
    f = pl.kernel(
        _sc_body,
        out_type=[
            jax.ShapeDtypeStruct((NG * NPAD, W), jnp.float32),
            jax.ShapeDtypeStruct((NG * NPAD, W), jnp.float32),
            jax.ShapeDtypeStruct((NG * NPAD, W), jnp.float32),
        ],
        mesh=mesh,
        compiler_params=_SC_PARAMS,
        scratch_types=[
            pltpu.VMEM_SHARED((NPAD, W), jnp.float32),
            pltpu.VMEM_SHARED((NPAD, W), jnp.float32),
            pltpu.VMEM_SHARED((NPAD, W), jnp.float32),
            pltpu.VMEM((CH,), jnp.int32),
            pltpu.VMEM((CH,), jnp.int32),
            pltpu.VMEM((CH,), jnp.int32),
            pltpu.VMEM((CH,), jnp.int32),
            pltpu.VMEM((CH, W), jnp.float32),
            pltpu.VMEM((CH, W), jnp.float32),
            pltpu.VMEM((CH, W), jnp.float32),
            pltpu.VMEM((CH, W), jnp.float32),
            pltpu.VMEM((CH, W), jnp.float32),
            pltpu.VMEM((CH, W), jnp.float32),
            pltpu.SemaphoreType.DMA,
            pltpu.SemaphoreType.DMA,
            pltpu.SemaphoreType.DMA,
            pltpu.SemaphoreType.DMA,
            pltpu.SemaphoreType.DMA,
            pltpu.SemaphoreType.DMA,
            pltpu.SemaphoreType.DMA,
            pltpu.SemaphoreType.DMA,
        ],
    )
    return f(src, dst, erel, xg0, xg1, xg2, xg3)


def _deg_body(dst_hbm, dego_hbm, degl, dstb):
    c = lax.axis_index("c")
    s = lax.axis_index("s")
    w = s * 2 + c

    zeros16 = jnp.zeros((16,), jnp.float32)
    ones16 = jnp.ones((16,), jnp.float32)

    def _deg_zero(t, _):
        degl[pl.ds(t * 16, 16)] = zeros16
        return _
    lax.fori_loop(0, NPAD // 16, _deg_zero, ())

    e_base = w * DEPS

    def _chunk_body(k, _):
        pltpu.sync_copy(dst_hbm.at[pl.ds(e_base + k * DCH, DCH)], dstb)

        def _acc(j, _i):
            idx16 = dstb[pl.ds(j * 16, 16)]
            plsc.addupdate_scatter(degl, [idx16], ones16)
            return _i
        lax.fori_loop(0, DCH // 16, _acc, ())
        return _
    lax.fori_loop(0, DNCH, _chunk_body, ())

    pltpu.sync_copy(degl, dego_hbm.at[pl.ds(w * NPAD, NPAD)])


def _scdeg(dst):
    mesh = plsc.VectorSubcoreMesh(core_axis_name="c", subcore_axis_name="s")
    f = pl.kernel(
        _deg_body,
        out_type=jax.ShapeDtypeStruct((NW * NPAD,), jnp.float32),
        mesh=mesh,
        compiler_params=_SC_PARAMS,
        scratch_types=[
            pltpu.VMEM((NPAD,), jnp.float32),
            pltpu.VMEM((DCH,), jnp.int32),
        ],
    )
    return f(dst)


def _post_body(e4_ref, w4_ref, p4_ref, dgp_ref, xsl_ref,
               w2_ref, b2_ref, wline_ref, bline_ref, out_ref):
    Se = jnp.concatenate([e4_ref[0], e4_ref[1], e4_ref[2], e4_ref[3]], axis=1)
    Sw = jnp.concatenate([w4_ref[0], w4_ref[1], w4_ref[2], w4_ref[3]], axis=1)
    Sp = jnp.concatenate([p4_ref[0], p4_ref[1], p4_ref[2], p4_ref[3]], axis=1)
    deg = jnp.sum(dgp_ref[...], axis=1)
    w2 = w2_ref[...]
    W2a = w2[0:D]
    W2b = w2[D:2 * D]
    W2c = w2[2 * D:3 * D]
    nei = (jnp.dot(Se, W2a + W2b, preferred_element_type=jnp.float32)
           + jnp.dot(Sw, W2a - W2b, preferred_element_type=jnp.float32)
           + jnp.dot(Sp, W2c, preferred_element_type=jnp.float32)
           + deg[:, None] * b2_ref[...])
    has_in = (deg > 0).astype(jnp.float32)[:, None]
    node = nei + xsl_ref[...] * has_in
    out = jnp.dot(node, wline_ref[...], preferred_element_type=jnp.float32) + bline_ref[...]
    out_ref[...] = jnp.maximum(out, 0.0)


def _post(e4, w4, p4, dgp, xsl, W2, b2r, Wline, bliner):
    B = 2000
    g = N_NODES // B
    return pl.pallas_call(
        _post_body,
        grid=(g,),
        in_specs=[
            pl.BlockSpec((NG, B, W), lambda i: (0, i, 0)),
            pl.BlockSpec((NG, B, W), lambda i: (0, i, 0)),
            pl.BlockSpec((NG, B, W), lambda i: (0, i, 0)),
            pl.BlockSpec((B, NW), lambda i: (i, 0)),
            pl.BlockSpec((B, D), lambda i: (i, 0)),
            pl.BlockSpec((3 * D, D), lambda i: (0, 0)),
            pl.BlockSpec((1, D), lambda i: (0, 0)),
            pl.BlockSpec((D, D), lambda i: (0, 0)),
            pl.BlockSpec((1, D), lambda i: (0, 0)),
        ],
        out_specs=pl.BlockSpec((B, D), lambda i: (i, 0)),
        out_shape=jax.ShapeDtypeStruct((N_NODES, D), jnp.float32),
    )(e4, w4, p4, dgp, xsl, W2, b2r, Wline, bliner)


def kernel(x, edge_index, norm, edge_rel_emd, target_rel_emd_new,
           W1, b1, W2, b2, Wsl, Wline, bline):
    del norm, target_rel_emd_new
    src = edge_index[0]
    dst = edge_index[1]
    xg0, xg1, xg2, xg3, xsl = _pre(x, W1, b1.reshape(1, D), Wsl)
    outE, outW, outP = _sc(src, dst, edge_rel_emd, xg0, xg1, xg2, xg3)
    degp = _scdeg(dst)
    e4 = outE.reshape(NG, NPAD, W)
    w4 = outW.reshape(NG, NPAD, W)
    p4 = outP.reshape(NG, NPAD, W)
    dgp = degp.reshape(NW, NPAD).T
    return _post(e4, w4, p4, dgp, xsl, W2, b2.reshape(1, D),
                 Wline, bline.reshape(1, D))


# speedup vs baseline: 3.5757x; 1.0021x over previous
import jax
import jax.numpy as jnp
from jax import lax
from jax.experimental import pallas as pl
from jax.experimental.pallas import tpu as pltpu
from jax.experimental.pallas import tpu_sc as plsc

N_NODES = 10000
N_EDGES = 320000
D = 128
W = 32
NG = 4
NPAD = 10016
NS = 16
NW = 32
EPS = N_EDGES // NS
CH = 80
NCH = EPS // CH
NSLOT = 4
NGROUPS = (NCH - 6) // NSLOT
ROWS_PER_SUB = NPAD // NS
DEPS = N_EDGES // NW
DCH = 80
DNCH = DEPS // DCH

_SC_PARAMS = pltpu.CompilerParams(use_tc_tiling_on_sc=False,
                                  needs_layout_passes=False)


def _pre_body(x_ref, w1_ref, b1_ref, wsl_ref,
              g0_ref, g1_ref, g2_ref, g3_ref, xsl_ref):
    xb = x_ref[...]
    h = jnp.dot(xb, w1_ref[...], preferred_element_type=jnp.float32) + b1_ref[...]
    g0_ref[...] = h[:, 0 * W:1 * W]
    g1_ref[...] = h[:, 1 * W:2 * W]
    g2_ref[...] = h[:, 2 * W:3 * W]
    g3_ref[...] = h[:, 3 * W:4 * W]
    xsl_ref[...] = jnp.dot(xb, wsl_ref[...], preferred_element_type=jnp.float32)


def _pre(x, W1, b1r, Wsl):
    B = 2000
    g = N_NODES // B
    return pl.pallas_call(
        _pre_body,
        grid=(g,),
        in_specs=[
            pl.BlockSpec((B, D), lambda i: (i, 0)),
            pl.BlockSpec((D, D), lambda i: (0, 0)),
            pl.BlockSpec((1, D), lambda i: (0, 0)),
            pl.BlockSpec((D, D), lambda i: (0, 0)),
        ],
        out_specs=[
            pl.BlockSpec((B, W), lambda i: (i, 0)),
            pl.BlockSpec((B, W), lambda i: (i, 0)),
            pl.BlockSpec((B, W), lambda i: (i, 0)),
            pl.BlockSpec((B, W), lambda i: (i, 0)),
            pl.BlockSpec((B, D), lambda i: (i, 0)),
        ],
        out_shape=[
            jax.ShapeDtypeStruct((N_NODES, W), jnp.float32),
            jax.ShapeDtypeStruct((N_NODES, W), jnp.float32),
            jax.ShapeDtypeStruct((N_NODES, W), jnp.float32),
            jax.ShapeDtypeStruct((N_NODES, W), jnp.float32),
            jax.ShapeDtypeStruct((N_NODES, D), jnp.float32),
        ],
    )(x, W1, b1r, Wsl)


def _sc_body(src_hbm, dst_hbm, erel_hbm, xg0_hbm, xg1_hbm, xg2_hbm, xg3_hbm,
             outE_hbm, outW_hbm, outP_hbm,
             accE, accW, accP,
             srcb0, srcb1, srcb2, srcb3, dstb0, dstb1, dstb2, dstb3,
             erelb0, erelb1, erelb2, erelb3, w1b0, w1b1, w1b2, w1b3,
             prodb0, prodb1, prodb2, prodb3,
             ses0, ses1, ses2, ses3, sed0, sed1, sed2, sed3,
             see0, see1, see2, see3, seg0, seg1, seg2, seg3,
             sesc0, sesc1, sesc2, sesc3):
    c = lax.axis_index("c")
    s = lax.axis_index("s")
    srcb = [srcb0, srcb1, srcb2, srcb3]
    dstb = [dstb0, dstb1, dstb2, dstb3]
    erelb = [erelb0, erelb1, erelb2, erelb3]
    w1b = [w1b0, w1b1, w1b2, w1b3]
    prodb = [prodb0, prodb1, prodb2, prodb3]
    ses = [ses0, ses1, ses2, ses3]
    sed = [sed0, sed1, sed2, sed3]
    see = [see0, see1, see2, see3]
    seg = [seg0, seg1, seg2, seg3]
    sesc = [sesc0, sesc1, sesc2, sesc3]

    zeros16 = jnp.zeros((16,), jnp.float32)
    r0 = s * ROWS_PER_SUB
    e_base = s * EPS

    def zero_strip():
        def _zb_zero(i, _):
            erelb0[i, pl.ds(0, 16)] = zeros16
            erelb0[i, pl.ds(16, 16)] = zeros16
            return _
        lax.fori_loop(0, CH, _zb_zero, ())
        left = ROWS_PER_SUB
        off = 0
        while left > 0:
            nb = min(CH, left)
            pltpu.sync_copy(erelb0.at[pl.ds(0, nb)], accE.at[pl.ds(r0 + off, nb)])
            pltpu.sync_copy(erelb0.at[pl.ds(0, nb)], accW.at[pl.ds(r0 + off, nb)])
            pltpu.sync_copy(erelb0.at[pl.ds(0, nb)], accP.at[pl.ds(r0 + off, nb)])
            off += nb
            left -= nb

    def phase(col0, xw1_hbm, out_base):
        zero_strip()
        plsc.subcore_barrier()

        def fire_inputs(k, b):
            e0 = e_base + k * CH
            pltpu.async_copy(src_hbm.at[pl.ds(e0, CH)], srcb[b], ses[b])
            pltpu.async_copy(dst_hbm.at[pl.ds(e0, CH)], dstb[b], sed[b])
            pltpu.async_copy(erel_hbm.at[pl.ds(e0, CH), pl.ds(col0, W)],
                             erelb[b], see[b])

        def wait_src(k, b):
            e0 = e_base + k * CH
            pltpu.make_async_copy(src_hbm.at[pl.ds(e0, CH)], srcb[b],
                                  ses[b]).wait()

        def fire_gather(b):
            pltpu.async_copy(xw1_hbm.at[srcb[b]], w1b[b], seg[b])

        def drain_scatters(b):
            pltpu.make_async_copy(erelb[b], accE.at[dstb[b]], sesc[b]).wait()
            pltpu.make_async_copy(w1b[b], accW.at[dstb[b]], sesc[b]).wait()
            pltpu.make_async_copy(prodb[b], accP.at[dstb[b]], sesc[b]).wait()

        def chunk_step(k, b, drain, fire2, gnext):
            e0 = e_base + k * CH
            pltpu.make_async_copy(xw1_hbm.at[srcb[b]], w1b[b], seg[b]).wait()
            pltpu.make_async_copy(erel_hbm.at[pl.ds(e0, CH), pl.ds(col0, W)],
                                  erelb[b], see[b]).wait()

            def _mul(i, _i):
                for r in range(2):
                    for j in range(0, W, 16):
                        prodb[b][2 * i + r, pl.ds(j, 16)] = (
                            erelb[b][2 * i + r, pl.ds(j, 16)]
                            * w1b[b][2 * i + r, pl.ds(j, 16)])
                return _i
            lax.fori_loop(0, CH // 2, _mul, ())

            pltpu.make_async_copy(dst_hbm.at[pl.ds(e0, CH)], dstb[b],
                                  sed[b]).wait()
            pltpu.async_copy(erelb[b], accE.at[dstb[b]], sesc[b], add=True)
            pltpu.async_copy(w1b[b], accW.at[dstb[b]], sesc[b], add=True)
            pltpu.async_copy(prodb[b], accP.at[dstb[b]], sesc[b], add=True)
            dsl = (b + 2) % NSLOT
            if drain:
                drain_scatters(dsl)
            if fire2:
                fire_inputs(k + 2, dsl)
            if gnext:
                nb = (b + 1) % NSLOT
                wait_src(k + 1, nb)
                fire_gather(nb)

        fire_inputs(0, 0)
        fire_inputs(1, 1)
        wait_src(0, 0)
        fire_gather(0)
        chunk_step(0, 0, False, True, True)
        chunk_step(1, 1, False, True, True)

        def _ring_body(g, _):
            for j in range(NSLOT):
                chunk_step(2 + NSLOT * g + j, (2 + j) % NSLOT, True, True, True)
            return _
        lax.fori_loop(0, NGROUPS, _ring_body, ())

        for k in range(2 + NSLOT * NGROUPS, NCH):
            chunk_step(k, k % NSLOT, True, k + 2 < NCH, k + 1 < NCH)
        drain_scatters((NCH - 2) % NSLOT)
        drain_scatters((NCH - 1) % NSLOT)

        plsc.subcore_barrier()
        out_r0 = out_base + r0
        pltpu.sync_copy(accE.at[pl.ds(r0, ROWS_PER_SUB)],
                        outE_hbm.at[pl.ds(out_r0, ROWS_PER_SUB)])
        pltpu.sync_copy(accW.at[pl.ds(r0, ROWS_PER_SUB)],
                        outW_hbm.at[pl.ds(out_r0, ROWS_PER_SUB)])
        pltpu.sync_copy(accP.at[pl.ds(r0, ROWS_PER_SUB)],
                        outP_hbm.at[pl.ds(out_r0, ROWS_PER_SUB)])

    def run_core(xw1s, gbase):
        for p in range(2):
            g = gbase + p
            phase(g * W, xw1s[p], g * NPAD)

    pl.when(c == 0)(lambda: run_core([xg0_hbm, xg1_hbm], 0))
    pl.when(c == 1)(lambda: run_core([xg2_hbm, xg3_hbm], 2))


def _sc(src, dst, erel, xg0, xg1, xg2, xg3):
    mesh = plsc.VectorSubcoreMesh(core_axis_name="c", subcore_axis_name="s")
    f = pl.kernel(
        _sc_body,
        out_type=[
            jax.ShapeDtypeStruct((NG * NPAD, W), jnp.float32),
            jax.ShapeDtypeStruct((NG * NPAD, W), jnp.float32),
            jax.ShapeDtypeStruct((NG * NPAD, W), jnp.float32),
        ],
        mesh=mesh,
        compiler_params=_SC_PARAMS,
        scratch_types=(
            [pltpu.VMEM_SHARED((NPAD, W), jnp.float32)] * 3
            + [pltpu.VMEM((CH,), jnp.int32)] * 8
            + [pltpu.VMEM((CH, W), jnp.float32)] * 12
            + [pltpu.SemaphoreType.DMA] * 20
        ),
    )
    return f(src, dst, erel, xg0, xg1, xg2, xg3)


def _deg_body(dst_hbm, dego_hbm, degl, dstb):
    c = lax.axis_index("c")
    s = lax.axis_index("s")
    w = s * 2 + c

    zeros16 = jnp.zeros((16,), jnp.float32)
    ones16 = jnp.ones((16,), jnp.float32)

    def _deg_zero(t, _):
        degl[pl.ds(t * 16, 16)] = zeros16
        return _
    lax.fori_loop(0, NPAD // 16, _deg_zero, ())

    e_base = w * DEPS

    def _chunk_body(k, _):
        pltpu.sync_copy(dst_hbm.at[pl.ds(e_base + k * DCH, DCH)], dstb)

        def _acc(j, _i):
            idx16 = dstb[pl.ds(j * 16, 16)]
            plsc.addupdate_scatter(degl, [idx16], ones16)
            return _i
        lax.fori_loop(0, DCH // 16, _acc, ())
        return _
    lax.fori_loop(0, DNCH, _chunk_body, ())

    pltpu.sync_copy(degl, dego_hbm.at[pl.ds(w * NPAD, NPAD)])


def _scdeg(dst):
    mesh = plsc.VectorSubcoreMesh(core_axis_name="c", subcore_axis_name="s")
    f = pl.kernel(
        _deg_body,
        out_type=jax.ShapeDtypeStruct((NW * NPAD,), jnp.float32),
        mesh=mesh,
        compiler_params=_SC_PARAMS,
        scratch_types=[
            pltpu.VMEM((NPAD,), jnp.float32),
            pltpu.VMEM((DCH,), jnp.int32),
        ],
    )
    return f(dst)


def _post_body(e4_ref, w4_ref, p4_ref, dgp_ref, xsl_ref,
               w2_ref, b2_ref, wline_ref, bline_ref, out_ref):
    Se = jnp.concatenate([e4_ref[0], e4_ref[1], e4_ref[2], e4_ref[3]], axis=1)
    Sw = jnp.concatenate([w4_ref[0], w4_ref[1], w4_ref[2], w4_ref[3]], axis=1)
    Sp = jnp.concatenate([p4_ref[0], p4_ref[1], p4_ref[2], p4_ref[3]], axis=1)
    deg = jnp.sum(dgp_ref[...], axis=1)
    w2 = w2_ref[...]
    W2a = w2[0:D]
    W2b = w2[D:2 * D]
    W2c = w2[2 * D:3 * D]
    nei = (jnp.dot(Se, W2a + W2b, preferred_element_type=jnp.float32)
           + jnp.dot(Sw, W2a - W2b, preferred_element_type=jnp.float32)
           + jnp.dot(Sp, W2c, preferred_element_type=jnp.float32)
           + deg[:, None] * b2_ref[...])
    has_in = (deg > 0).astype(jnp.float32)[:, None]
    node = nei + xsl_ref[...] * has_in
    out = jnp.dot(node, wline_ref[...], preferred_element_type=jnp.float32) + bline_ref[...]
    out_ref[...] = jnp.maximum(out, 0.0)


def _post(e4, w4, p4, dgp, xsl, W2, b2r, Wline, bliner):
    B = 2000
    g = N_NODES // B
    return pl.pallas_call(
        _post_body,
        grid=(g,),
        in_specs=[
            pl.BlockSpec((NG, B, W), lambda i: (0, i, 0)),
            pl.BlockSpec((NG, B, W), lambda i: (0, i, 0)),
            pl.BlockSpec((NG, B, W), lambda i: (0, i, 0)),
            pl.BlockSpec((B, NW), lambda i: (i, 0)),
            pl.BlockSpec((B, D), lambda i: (i, 0)),
            pl.BlockSpec((3 * D, D), lambda i: (0, 0)),
            pl.BlockSpec((1, D), lambda i: (0, 0)),
            pl.BlockSpec((D, D), lambda i: (0, 0)),
            pl.BlockSpec((1, D), lambda i: (0, 0)),
        ],
        out_specs=pl.BlockSpec((B, D), lambda i: (i, 0)),
        out_shape=jax.ShapeDtypeStruct((N_NODES, D), jnp.float32),
    )(e4, w4, p4, dgp, xsl, W2, b2r, Wline, bliner)


def kernel(x, edge_index, norm, edge_rel_emd, target_rel_emd_new,
           W1, b1, W2, b2, Wsl, Wline, bline):
    del norm, target_rel_emd_new
    src = edge_index[0]
    dst = edge_index[1]
    xg0, xg1, xg2, xg3, xsl = _pre(x, W1, b1.reshape(1, D), Wsl)
    outE, outW, outP = _sc(src, dst, edge_rel_emd, xg0, xg1, xg2, xg3)
    degp = _scdeg(dst)
    e4 = outE.reshape(NG, NPAD, W)
    w4 = outW.reshape(NG, NPAD, W)
    p4 = outP.reshape(NG, NPAD, W)
    dgp = degp.reshape(NW, NPAD).T
    return _post(e4, w4, p4, dgp, xsl, W2, b2.reshape(1, D),
                 Wline, bline.reshape(1, D))


# speedup vs baseline: 4.3168x; 1.2073x over previous
import jax
import jax.numpy as jnp
from jax import lax
from jax.experimental import pallas as pl
from jax.experimental.pallas import tpu as pltpu
from jax.experimental.pallas import tpu_sc as plsc

N_NODES = 10000
N_EDGES = 320000
D = 128
W = 32
NG = 4
NPAD = 10016
NS = 16
NW = 32
EPS = N_EDGES // NS
CH = 160
NCH = EPS // CH
NSLOT = 4
NGROUPS = (NCH - 6) // NSLOT
ROWS_PER_SUB = NPAD // NS
DEPS = N_EDGES // NW
DCH = 80
DNCH = DEPS // DCH

_SC_PARAMS = pltpu.CompilerParams(use_tc_tiling_on_sc=False,
                                  needs_layout_passes=False)


def _pre_body(x_ref, w1_ref, b1_ref, wsl_ref,
              g0_ref, g1_ref, g2_ref, g3_ref, xsl_ref):
    xb = x_ref[...]
    h = jnp.dot(xb, w1_ref[...], preferred_element_type=jnp.float32) + b1_ref[...]
    g0_ref[...] = h[:, 0 * W:1 * W]
    g1_ref[...] = h[:, 1 * W:2 * W]
    g2_ref[...] = h[:, 2 * W:3 * W]
    g3_ref[...] = h[:, 3 * W:4 * W]
    xsl_ref[...] = jnp.dot(xb, wsl_ref[...], preferred_element_type=jnp.float32)


def _pre(x, W1, b1r, Wsl):
    B = 2000
    g = N_NODES // B
    return pl.pallas_call(
        _pre_body,
        grid=(g,),
        in_specs=[
            pl.BlockSpec((B, D), lambda i: (i, 0)),
            pl.BlockSpec((D, D), lambda i: (0, 0)),
            pl.BlockSpec((1, D), lambda i: (0, 0)),
            pl.BlockSpec((D, D), lambda i: (0, 0)),
        ],
        out_specs=[
            pl.BlockSpec((B, W), lambda i: (i, 0)),
            pl.BlockSpec((B, W), lambda i: (i, 0)),
            pl.BlockSpec((B, W), lambda i: (i, 0)),
            pl.BlockSpec((B, W), lambda i: (i, 0)),
            pl.BlockSpec((B, D), lambda i: (i, 0)),
        ],
        out_shape=[
            jax.ShapeDtypeStruct((N_NODES, W), jnp.float32),
            jax.ShapeDtypeStruct((N_NODES, W), jnp.float32),
            jax.ShapeDtypeStruct((N_NODES, W), jnp.float32),
            jax.ShapeDtypeStruct((N_NODES, W), jnp.float32),
            jax.ShapeDtypeStruct((N_NODES, D), jnp.float32),
        ],
    )(x, W1, b1r, Wsl)


def _sc_body(src_hbm, dst_hbm, erel_hbm, xg0_hbm, xg1_hbm, xg2_hbm, xg3_hbm,
             outE_hbm, outW_hbm, outP_hbm,
             accE, accW, accP,
             srcb0, srcb1, srcb2, srcb3, dstb0, dstb1, dstb2, dstb3,
             erelb0, erelb1, erelb2, erelb3, w1b0, w1b1, w1b2, w1b3,
             prodb0, prodb1, prodb2, prodb3,
             ses0, ses1, ses2, ses3, sed0, sed1, sed2, sed3,
             see0, see1, see2, see3, seg0, seg1, seg2, seg3,
             sesc0, sesc1, sesc2, sesc3):
    c = lax.axis_index("c")
    s = lax.axis_index("s")
    srcb = [srcb0, srcb1, srcb2, srcb3]
    dstb = [dstb0, dstb1, dstb2, dstb3]
    erelb = [erelb0, erelb1, erelb2, erelb3]
    w1b = [w1b0, w1b1, w1b2, w1b3]
    prodb = [prodb0, prodb1, prodb2, prodb3]
    ses = [ses0, ses1, ses2, ses3]
    sed = [sed0, sed1, sed2, sed3]
    see = [see0, see1, see2, see3]
    seg = [seg0, seg1, seg2, seg3]
    sesc = [sesc0, sesc1, sesc2, sesc3]

    zeros16 = jnp.zeros((16,), jnp.float32)
    r0 = s * ROWS_PER_SUB
    e_base = s * EPS

    def zero_strip():
        def _zb_zero(i, _):
            erelb0[i, pl.ds(0, 16)] = zeros16
            erelb0[i, pl.ds(16, 16)] = zeros16
            return _
        lax.fori_loop(0, CH, _zb_zero, ())
        left = ROWS_PER_SUB
        off = 0
        while left > 0:
            nb = min(CH, left)
            pltpu.sync_copy(erelb0.at[pl.ds(0, nb)], accE.at[pl.ds(r0 + off, nb)])
            pltpu.sync_copy(erelb0.at[pl.ds(0, nb)], accW.at[pl.ds(r0 + off, nb)])
            pltpu.sync_copy(erelb0.at[pl.ds(0, nb)], accP.at[pl.ds(r0 + off, nb)])
            off += nb
            left -= nb

    def phase(col0, xw1_hbm, out_base):
        zero_strip()
        plsc.subcore_barrier()

        def fire_inputs(k, b):
            e0 = e_base + k * CH
            pltpu.async_copy(src_hbm.at[pl.ds(e0, CH)], srcb[b], ses[b])
            pltpu.async_copy(dst_hbm.at[pl.ds(e0, CH)], dstb[b], sed[b])
            pltpu.async_copy(erel_hbm.at[pl.ds(e0, CH), pl.ds(col0, W)],
                             erelb[b], see[b])

        def wait_src(k, b):
            e0 = e_base + k * CH
            pltpu.make_async_copy(src_hbm.at[pl.ds(e0, CH)], srcb[b],
                                  ses[b]).wait()

        def fire_gather(b):
            pltpu.async_copy(xw1_hbm.at[srcb[b]], w1b[b], seg[b])

        def drain_scatters(b):
            pltpu.make_async_copy(erelb[b], accE.at[dstb[b]], sesc[b]).wait()
            pltpu.make_async_copy(w1b[b], accW.at[dstb[b]], sesc[b]).wait()
            pltpu.make_async_copy(prodb[b], accP.at[dstb[b]], sesc[b]).wait()

        def chunk_step(k, b, drain, fire2, gnext):
            e0 = e_base + k * CH
            pltpu.make_async_copy(xw1_hbm.at[srcb[b]], w1b[b], seg[b]).wait()
            pltpu.make_async_copy(erel_hbm.at[pl.ds(e0, CH), pl.ds(col0, W)],
                                  erelb[b], see[b]).wait()

            def _mul(i, _i):
                for r in range(2):
                    for j in range(0, W, 16):
                        prodb[b][2 * i + r, pl.ds(j, 16)] = (
                            erelb[b][2 * i + r, pl.ds(j, 16)]
                            * w1b[b][2 * i + r, pl.ds(j, 16)])
                return _i
            lax.fori_loop(0, CH // 2, _mul, ())

            pltpu.make_async_copy(dst_hbm.at[pl.ds(e0, CH)], dstb[b],
                                  sed[b]).wait()
            pltpu.async_copy(erelb[b], accE.at[dstb[b]], sesc[b], add=True)
            pltpu.async_copy(w1b[b], accW.at[dstb[b]], sesc[b], add=True)
            pltpu.async_copy(prodb[b], accP.at[dstb[b]], sesc[b], add=True)
            dsl = (b + 2) % NSLOT
            if drain:
                drain_scatters(dsl)
            if fire2:
                fire_inputs(k + 2, dsl)
            if gnext:
                nb = (b + 1) % NSLOT
                wait_src(k + 1, nb)
                fire_gather(nb)

        fire_inputs(0, 0)
        fire_inputs(1, 1)
        wait_src(0, 0)
        fire_gather(0)
        chunk_step(0, 0, False, True, True)
        chunk_step(1, 1, False, True, True)

        def _ring_body(g, _):
            for j in range(NSLOT):
                chunk_step(2 + NSLOT * g + j, (2 + j) % NSLOT, True, True, True)
            return _
        lax.fori_loop(0, NGROUPS, _ring_body, ())

        for k in range(2 + NSLOT * NGROUPS, NCH):
            chunk_step(k, k % NSLOT, True, k + 2 < NCH, k + 1 < NCH)
        drain_scatters((NCH - 2) % NSLOT)
        drain_scatters((NCH - 1) % NSLOT)

        plsc.subcore_barrier()
        out_r0 = out_base + r0
        pltpu.sync_copy(accE.at[pl.ds(r0, ROWS_PER_SUB)],
                        outE_hbm.at[pl.ds(out_r0, ROWS_PER_SUB)])
        pltpu.sync_copy(accW.at[pl.ds(r0, ROWS_PER_SUB)],
                        outW_hbm.at[pl.ds(out_r0, ROWS_PER_SUB)])
        pltpu.sync_copy(accP.at[pl.ds(r0, ROWS_PER_SUB)],
                        outP_hbm.at[pl.ds(out_r0, ROWS_PER_SUB)])

    def run_core(xw1s, gbase):
        for p in range(2):
            g = gbase + p
            phase(g * W, xw1s[p], g * NPAD)

    pl.when(c == 0)(lambda: run_core([xg0_hbm, xg1_hbm], 0))
    pl.when(c == 1)(lambda: run_core([xg2_hbm, xg3_hbm], 2))


def _sc(src, dst, erel, xg0, xg1, xg2, xg3):
    mesh = plsc.VectorSubcoreMesh(core_axis_name="c", subcore_axis_name="s")
    f = pl.kernel(
        _sc_body,
        out_type=[
            jax.ShapeDtypeStruct((NG * NPAD, W), jnp.float32),
            jax.ShapeDtypeStruct((NG * NPAD, W), jnp.float32),
            jax.ShapeDtypeStruct((NG * NPAD, W), jnp.float32),
        ],
        mesh=mesh,
        compiler_params=_SC_PARAMS,
        scratch_types=(
            [pltpu.VMEM_SHARED((NPAD, W), jnp.float32)] * 3
            + [pltpu.VMEM((CH,), jnp.int32)] * 8
            + [pltpu.VMEM((CH, W), jnp.float32)] * 12
            + [pltpu.SemaphoreType.DMA] * 20
        ),
    )
    return f(src, dst, erel, xg0, xg1, xg2, xg3)


def _deg_body(dst_hbm, dego_hbm, degl, dstb):
    c = lax.axis_index("c")
    s = lax.axis_index("s")
    w = s * 2 + c

    zeros16 = jnp.zeros((16,), jnp.float32)
    ones16 = jnp.ones((16,), jnp.float32)

    def _deg_zero(t, _):
        degl[pl.ds(t * 16, 16)] = zeros16
        return _
    lax.fori_loop(0, NPAD // 16, _deg_zero, ())

    e_base = w * DEPS

    def _chunk_body(k, _):
        pltpu.sync_copy(dst_hbm.at[pl.ds(e_base + k * DCH, DCH)], dstb)

        def _acc(j, _i):
            idx16 = dstb[pl.ds(j * 16, 16)]
            plsc.addupdate_scatter(degl, [idx16], ones16)
            return _i
        lax.fori_loop(0, DCH // 16, _acc, ())
        return _
    lax.fori_loop(0, DNCH, _chunk_body, ())

    pltpu.sync_copy(degl, dego_hbm.at[pl.ds(w * NPAD, NPAD)])


def _scdeg(dst):
    mesh = plsc.VectorSubcoreMesh(core_axis_name="c", subcore_axis_name="s")
    f = pl.kernel(
        _deg_body,
        out_type=jax.ShapeDtypeStruct((NW * NPAD,), jnp.float32),
        mesh=mesh,
        compiler_params=_SC_PARAMS,
        scratch_types=[
            pltpu.VMEM((NPAD,), jnp.float32),
            pltpu.VMEM((DCH,), jnp.int32),
        ],
    )
    return f(dst)


def _post_body(e4_ref, w4_ref, p4_ref, dgp_ref, xsl_ref,
               w2_ref, b2_ref, wline_ref, bline_ref, out_ref):
    Se = jnp.concatenate([e4_ref[0], e4_ref[1], e4_ref[2], e4_ref[3]], axis=1)
    Sw = jnp.concatenate([w4_ref[0], w4_ref[1], w4_ref[2], w4_ref[3]], axis=1)
    Sp = jnp.concatenate([p4_ref[0], p4_ref[1], p4_ref[2], p4_ref[3]], axis=1)
    deg = jnp.sum(dgp_ref[...], axis=1)
    w2 = w2_ref[...]
    W2a = w2[0:D]
    W2b = w2[D:2 * D]
    W2c = w2[2 * D:3 * D]
    nei = (jnp.dot(Se, W2a + W2b, preferred_element_type=jnp.float32)
           + jnp.dot(Sw, W2a - W2b, preferred_element_type=jnp.float32)
           + jnp.dot(Sp, W2c, preferred_element_type=jnp.float32)
           + deg[:, None] * b2_ref[...])
    has_in = (deg > 0).astype(jnp.float32)[:, None]
    node = nei + xsl_ref[...] * has_in
    out = jnp.dot(node, wline_ref[...], preferred_element_type=jnp.float32) + bline_ref[...]
    out_ref[...] = jnp.maximum(out, 0.0)


def _post(e4, w4, p4, dgp, xsl, W2, b2r, Wline, bliner):
    B = 2000
    g = N_NODES // B
    return pl.pallas_call(
        _post_body,
        grid=(g,),
        in_specs=[
            pl.BlockSpec((NG, B, W), lambda i: (0, i, 0)),
            pl.BlockSpec((NG, B, W), lambda i: (0, i, 0)),
            pl.BlockSpec((NG, B, W), lambda i: (0, i, 0)),
            pl.BlockSpec((B, NW), lambda i: (i, 0)),
            pl.BlockSpec((B, D), lambda i: (i, 0)),
            pl.BlockSpec((3 * D, D), lambda i: (0, 0)),
            pl.BlockSpec((1, D), lambda i: (0, 0)),
            pl.BlockSpec((D, D), lambda i: (0, 0)),
            pl.BlockSpec((1, D), lambda i: (0, 0)),
        ],
        out_specs=pl.BlockSpec((B, D), lambda i: (i, 0)),
        out_shape=jax.ShapeDtypeStruct((N_NODES, D), jnp.float32),
    )(e4, w4, p4, dgp, xsl, W2, b2r, Wline, bliner)


def kernel(x, edge_index, norm, edge_rel_emd, target_rel_emd_new,
           W1, b1, W2, b2, Wsl, Wline, bline):
    del norm, target_rel_emd_new
    src = edge_index[0]
    dst = edge_index[1]
    xg0, xg1, xg2, xg3, xsl = _pre(x, W1, b1.reshape(1, D), Wsl)
    outE, outW, outP = _sc(src, dst, edge_rel_emd, xg0, xg1, xg2, xg3)
    degp = _scdeg(dst)
    e4 = outE.reshape(NG, NPAD, W)
    w4 = outW.reshape(NG, NPAD, W)
    p4 = outP.reshape(NG, NPAD, W)
    dgp = degp.reshape(NW, NPAD).T
    return _post(e4, w4, p4, dgp, xsl, W2, b2.reshape(1, D),
                 Wline, bline.reshape(1, D))


# speedup vs baseline: 4.5603x; 1.0564x over previous
import jax
import jax.numpy as jnp
from jax import lax
from jax.experimental import pallas as pl
from jax.experimental.pallas import tpu as pltpu
from jax.experimental.pallas import tpu_sc as plsc

N_NODES = 10000
N_EDGES = 320000
D = 128
W = 32
NG = 4
NPAD = 10016
NS = 16
NW = 32
EPS = N_EDGES // NS
CH = 160
NCH = EPS // CH
NSLOT = 4
NGROUPS = (NCH - 6) // NSLOT
ROWS_PER_SUB = NPAD // NS
DEPS = N_EDGES // NW
DCH = 400
DNCH = DEPS // DCH

_SC_PARAMS = pltpu.CompilerParams(use_tc_tiling_on_sc=False,
                                  needs_layout_passes=False)


def _pre_body(x_ref, w1_ref, b1_ref, wsl_ref,
              g0_ref, g1_ref, g2_ref, g3_ref, xsl_ref):
    xb = x_ref[...]
    h = jnp.dot(xb, w1_ref[...], preferred_element_type=jnp.float32) + b1_ref[...]
    g0_ref[...] = h[:, 0 * W:1 * W]
    g1_ref[...] = h[:, 1 * W:2 * W]
    g2_ref[...] = h[:, 2 * W:3 * W]
    g3_ref[...] = h[:, 3 * W:4 * W]
    xsl_ref[...] = jnp.dot(xb, wsl_ref[...], preferred_element_type=jnp.float32)


def _pre(x, W1, b1r, Wsl):
    B = 2000
    g = N_NODES // B
    return pl.pallas_call(
        _pre_body,
        grid=(g,),
        in_specs=[
            pl.BlockSpec((B, D), lambda i: (i, 0)),
            pl.BlockSpec((D, D), lambda i: (0, 0)),
            pl.BlockSpec((1, D), lambda i: (0, 0)),
            pl.BlockSpec((D, D), lambda i: (0, 0)),
        ],
        out_specs=[
            pl.BlockSpec((B, W), lambda i: (i, 0)),
            pl.BlockSpec((B, W), lambda i: (i, 0)),
            pl.BlockSpec((B, W), lambda i: (i, 0)),
            pl.BlockSpec((B, W), lambda i: (i, 0)),
            pl.BlockSpec((B, D), lambda i: (i, 0)),
        ],
        out_shape=[
            jax.ShapeDtypeStruct((N_NODES, W), jnp.float32),
            jax.ShapeDtypeStruct((N_NODES, W), jnp.float32),
            jax.ShapeDtypeStruct((N_NODES, W), jnp.float32),
            jax.ShapeDtypeStruct((N_NODES, W), jnp.float32),
            jax.ShapeDtypeStruct((N_NODES, D), jnp.float32),
        ],
    )(x, W1, b1r, Wsl)


def _sc_body(src_hbm, dst_hbm, erel_hbm, xg0_hbm, xg1_hbm, xg2_hbm, xg3_hbm,
             outE_hbm, outW_hbm, outP_hbm,
             accE, accW, accP,
             srcb0, srcb1, srcb2, srcb3, dstb0, dstb1, dstb2, dstb3,
             erelb0, erelb1, erelb2, erelb3, w1b0, w1b1, w1b2, w1b3,
             prodb0, prodb1, prodb2, prodb3,
             ses0, ses1, ses2, ses3, sed0, sed1, sed2, sed3,
             see0, see1, see2, see3, seg0, seg1, seg2, seg3,
             sesc0, sesc1, sesc2, sesc3):
    c = lax.axis_index("c")
    s = lax.axis_index("s")
    srcb = [srcb0, srcb1, srcb2, srcb3]
    dstb = [dstb0, dstb1, dstb2, dstb3]
    erelb = [erelb0, erelb1, erelb2, erelb3]
    w1b = [w1b0, w1b1, w1b2, w1b3]
    prodb = [prodb0, prodb1, prodb2, prodb3]
    ses = [ses0, ses1, ses2, ses3]
    sed = [sed0, sed1, sed2, sed3]
    see = [see0, see1, see2, see3]
    seg = [seg0, seg1, seg2, seg3]
    sesc = [sesc0, sesc1, sesc2, sesc3]

    zeros16 = jnp.zeros((16,), jnp.float32)
    r0 = s * ROWS_PER_SUB
    e_base = s * EPS

    def zero_strip():
        def _zb_zero(i, _):
            erelb0[i, pl.ds(0, 16)] = zeros16
            erelb0[i, pl.ds(16, 16)] = zeros16
            return _
        lax.fori_loop(0, CH, _zb_zero, ())
        left = ROWS_PER_SUB
        off = 0
        while left > 0:
            nb = min(CH, left)
            pltpu.sync_copy(erelb0.at[pl.ds(0, nb)], accE.at[pl.ds(r0 + off, nb)])
            pltpu.sync_copy(erelb0.at[pl.ds(0, nb)], accW.at[pl.ds(r0 + off, nb)])
            pltpu.sync_copy(erelb0.at[pl.ds(0, nb)], accP.at[pl.ds(r0 + off, nb)])
            off += nb
            left -= nb

    def phase(col0, xw1_hbm, out_base):
        zero_strip()
        plsc.subcore_barrier()

        def fire_inputs(k, b):
            e0 = e_base + k * CH
            pltpu.async_copy(src_hbm.at[pl.ds(e0, CH)], srcb[b], ses[b])
            pltpu.async_copy(dst_hbm.at[pl.ds(e0, CH)], dstb[b], sed[b])
            pltpu.async_copy(erel_hbm.at[pl.ds(e0, CH), pl.ds(col0, W)],
                             erelb[b], see[b])

        def wait_src(k, b):
            e0 = e_base + k * CH
            pltpu.make_async_copy(src_hbm.at[pl.ds(e0, CH)], srcb[b],
                                  ses[b]).wait()

        def fire_gather(b):
            pltpu.async_copy(xw1_hbm.at[srcb[b]], w1b[b], seg[b])

        def drain_scatters(b):
            pltpu.make_async_copy(erelb[b], accE.at[dstb[b]], sesc[b]).wait()
            pltpu.make_async_copy(w1b[b], accW.at[dstb[b]], sesc[b]).wait()
            pltpu.make_async_copy(prodb[b], accP.at[dstb[b]], sesc[b]).wait()

        def chunk_step(k, b, drain, fire2, gnext):
            e0 = e_base + k * CH
            pltpu.make_async_copy(xw1_hbm.at[srcb[b]], w1b[b], seg[b]).wait()
            pltpu.make_async_copy(erel_hbm.at[pl.ds(e0, CH), pl.ds(col0, W)],
                                  erelb[b], see[b]).wait()

            def _mul(i, _i):
                for r in range(2):
                    for j in range(0, W, 16):
                        prodb[b][2 * i + r, pl.ds(j, 16)] = (
                            erelb[b][2 * i + r, pl.ds(j, 16)]
                            * w1b[b][2 * i + r, pl.ds(j, 16)])
                return _i
            lax.fori_loop(0, CH // 2, _mul, ())

            pltpu.make_async_copy(dst_hbm.at[pl.ds(e0, CH)], dstb[b],
                                  sed[b]).wait()
            pltpu.async_copy(erelb[b], accE.at[dstb[b]], sesc[b], add=True)
            pltpu.async_copy(w1b[b], accW.at[dstb[b]], sesc[b], add=True)
            pltpu.async_copy(prodb[b], accP.at[dstb[b]], sesc[b], add=True)
            dsl = (b + 2) % NSLOT
            if drain:
                drain_scatters(dsl)
            if fire2:
                fire_inputs(k + 2, dsl)
            if gnext:
                nb = (b + 1) % NSLOT
                wait_src(k + 1, nb)
                fire_gather(nb)

        fire_inputs(0, 0)
        fire_inputs(1, 1)
        wait_src(0, 0)
        fire_gather(0)
        chunk_step(0, 0, False, True, True)
        chunk_step(1, 1, False, True, True)

        def _ring_body(g, _):
            for j in range(NSLOT):
                chunk_step(2 + NSLOT * g + j, (2 + j) % NSLOT, True, True, True)
            return _
        lax.fori_loop(0, NGROUPS, _ring_body, ())

        for k in range(2 + NSLOT * NGROUPS, NCH):
            chunk_step(k, k % NSLOT, True, k + 2 < NCH, k + 1 < NCH)
        drain_scatters((NCH - 2) % NSLOT)
        drain_scatters((NCH - 1) % NSLOT)

        plsc.subcore_barrier()
        out_r0 = out_base + r0
        pltpu.sync_copy(accE.at[pl.ds(r0, ROWS_PER_SUB)],
                        outE_hbm.at[pl.ds(out_r0, ROWS_PER_SUB)])
        pltpu.sync_copy(accW.at[pl.ds(r0, ROWS_PER_SUB)],
                        outW_hbm.at[pl.ds(out_r0, ROWS_PER_SUB)])
        pltpu.sync_copy(accP.at[pl.ds(r0, ROWS_PER_SUB)],
                        outP_hbm.at[pl.ds(out_r0, ROWS_PER_SUB)])

    def run_core(xw1s, gbase):
        for p in range(2):
            g = gbase + p
            phase(g * W, xw1s[p], g * NPAD)

    pl.when(c == 0)(lambda: run_core([xg0_hbm, xg1_hbm], 0))
    pl.when(c == 1)(lambda: run_core([xg2_hbm, xg3_hbm], 2))


def _sc(src, dst, erel, xg0, xg1, xg2, xg3):
    mesh = plsc.VectorSubcoreMesh(core_axis_name="c", subcore_axis_name="s")
    f = pl.kernel(
        _sc_body,
        out_type=[
            jax.ShapeDtypeStruct((NG * NPAD, W), jnp.float32),
            jax.ShapeDtypeStruct((NG * NPAD, W), jnp.float32),
            jax.ShapeDtypeStruct((NG * NPAD, W), jnp.float32),
        ],
        mesh=mesh,
        compiler_params=_SC_PARAMS,
        scratch_types=(
            [pltpu.VMEM_SHARED((NPAD, W), jnp.float32)] * 3
            + [pltpu.VMEM((CH,), jnp.int32)] * 8
            + [pltpu.VMEM((CH, W), jnp.float32)] * 12
            + [pltpu.SemaphoreType.DMA] * 20
        ),
    )
    return f(src, dst, erel, xg0, xg1, xg2, xg3)


def _deg_body(dst_hbm, dego_hbm, degl, dstb):
    c = lax.axis_index("c")
    s = lax.axis_index("s")
    w = s * 2 + c

    zeros16 = jnp.zeros((16,), jnp.float32)
    ones16 = jnp.ones((16,), jnp.float32)

    def _deg_zero(t, _):
        degl[pl.ds(t * 16, 16)] = zeros16
        return _
    lax.fori_loop(0, NPAD // 16, _deg_zero, ())

    e_base = w * DEPS

    def _chunk_body(k, _):
        pltpu.sync_copy(dst_hbm.at[pl.ds(e_base + k * DCH, DCH)], dstb)

        def _acc(j, _i):
            idx16 = dstb[pl.ds(j * 16, 16)]
            plsc.addupdate_scatter(degl, [idx16], ones16)
            return _i
        lax.fori_loop(0, DCH // 16, _acc, ())
        return _
    lax.fori_loop(0, DNCH, _chunk_body, ())

    pltpu.sync_copy(degl, dego_hbm.at[pl.ds(w * NPAD, NPAD)])


def _scdeg(dst):
    mesh = plsc.VectorSubcoreMesh(core_axis_name="c", subcore_axis_name="s")
    f = pl.kernel(
        _deg_body,
        out_type=jax.ShapeDtypeStruct((NW * NPAD,), jnp.float32),
        mesh=mesh,
        compiler_params=_SC_PARAMS,
        scratch_types=[
            pltpu.VMEM((NPAD,), jnp.float32),
            pltpu.VMEM((DCH,), jnp.int32),
        ],
    )
    return f(dst)


def _post_body(e4_ref, w4_ref, p4_ref, dgp_ref, xsl_ref,
               w2_ref, b2_ref, wline_ref, bline_ref, out_ref):
    Se = jnp.concatenate([e4_ref[0], e4_ref[1], e4_ref[2], e4_ref[3]], axis=1)
    Sw = jnp.concatenate([w4_ref[0], w4_ref[1], w4_ref[2], w4_ref[3]], axis=1)
    Sp = jnp.concatenate([p4_ref[0], p4_ref[1], p4_ref[2], p4_ref[3]], axis=1)
    deg = jnp.sum(dgp_ref[...], axis=1)
    w2 = w2_ref[...]
    W2a = w2[0:D]
    W2b = w2[D:2 * D]
    W2c = w2[2 * D:3 * D]
    nei = (jnp.dot(Se, W2a + W2b, preferred_element_type=jnp.float32)
           + jnp.dot(Sw, W2a - W2b, preferred_element_type=jnp.float32)
           + jnp.dot(Sp, W2c, preferred_element_type=jnp.float32)
           + deg[:, None] * b2_ref[...])
    has_in = (deg > 0).astype(jnp.float32)[:, None]
    node = nei + xsl_ref[...] * has_in
    out = jnp.dot(node, wline_ref[...], preferred_element_type=jnp.float32) + bline_ref[...]
    out_ref[...] = jnp.maximum(out, 0.0)


def _post(e4, w4, p4, dgp, xsl, W2, b2r, Wline, bliner):
    B = 2000
    g = N_NODES // B
    return pl.pallas_call(
        _post_body,
        grid=(g,),
        in_specs=[
            pl.BlockSpec((NG, B, W), lambda i: (0, i, 0)),
            pl.BlockSpec((NG, B, W), lambda i: (0, i, 0)),
            pl.BlockSpec((NG, B, W), lambda i: (0, i, 0)),
            pl.BlockSpec((B, NW), lambda i: (i, 0)),
            pl.BlockSpec((B, D), lambda i: (i, 0)),
            pl.BlockSpec((3 * D, D), lambda i: (0, 0)),
            pl.BlockSpec((1, D), lambda i: (0, 0)),
            pl.BlockSpec((D, D), lambda i: (0, 0)),
            pl.BlockSpec((1, D), lambda i: (0, 0)),
        ],
        out_specs=pl.BlockSpec((B, D), lambda i: (i, 0)),
        out_shape=jax.ShapeDtypeStruct((N_NODES, D), jnp.float32),
    )(e4, w4, p4, dgp, xsl, W2, b2r, Wline, bliner)


def kernel(x, edge_index, norm, edge_rel_emd, target_rel_emd_new,
           W1, b1, W2, b2, Wsl, Wline, bline):
    del norm, target_rel_emd_new
    src = edge_index[0]
    dst = edge_index[1]
    xg0, xg1, xg2, xg3, xsl = _pre(x, W1, b1.reshape(1, D), Wsl)
    outE, outW, outP = _sc(src, dst, edge_rel_emd, xg0, xg1, xg2, xg3)
    degp = _scdeg(dst)
    e4 = outE.reshape(NG, NPAD, W)
    w4 = outW.reshape(NG, NPAD, W)
    p4 = outP.reshape(NG, NPAD, W)
    dgp = degp.reshape(NW, NPAD).T
    return _post(e4, w4, p4, dgp, xsl, W2, b2.reshape(1, D),
                 Wline, bline.reshape(1, D))


# speedup vs baseline: 4.7192x; 1.0348x over previous
import jax
import jax.numpy as jnp
from jax import lax
from jax.experimental import pallas as pl
from jax.experimental.pallas import tpu as pltpu
from jax.experimental.pallas import tpu_sc as plsc

N_NODES = 10000
N_EDGES = 320000
D = 128
W = 32
NG = 4
NPAD = 10016
NS = 16
NW = 32
EPS = N_EDGES // NS
CH = 200
NCH = EPS // CH
NSLOT = 3
NGROUPS = (NCH - 6) // NSLOT
ROWS_PER_SUB = NPAD // NS
DEPS = N_EDGES // NW
DCH = 400
DNCH = DEPS // DCH

_SC_PARAMS = pltpu.CompilerParams(use_tc_tiling_on_sc=False,
                                  needs_layout_passes=False)


def _pre_body(x_ref, w1_ref, b1_ref, wsl_ref,
              g0_ref, g1_ref, g2_ref, g3_ref, xsl_ref):
    xb = x_ref[...]
    h = jnp.dot(xb, w1_ref[...], preferred_element_type=jnp.float32) + b1_ref[...]
    g0_ref[...] = h[:, 0 * W:1 * W]
    g1_ref[...] = h[:, 1 * W:2 * W]
    g2_ref[...] = h[:, 2 * W:3 * W]
    g3_ref[...] = h[:, 3 * W:4 * W]
    xsl_ref[...] = jnp.dot(xb, wsl_ref[...], preferred_element_type=jnp.float32)


def _pre(x, W1, b1r, Wsl):
    B = 2000
    g = N_NODES // B
    return pl.pallas_call(
        _pre_body,
        grid=(g,),
        in_specs=[
            pl.BlockSpec((B, D), lambda i: (i, 0)),
            pl.BlockSpec((D, D), lambda i: (0, 0)),
            pl.BlockSpec((1, D), lambda i: (0, 0)),
            pl.BlockSpec((D, D), lambda i: (0, 0)),
        ],
        out_specs=[
            pl.BlockSpec((B, W), lambda i: (i, 0)),
            pl.BlockSpec((B, W), lambda i: (i, 0)),
            pl.BlockSpec((B, W), lambda i: (i, 0)),
            pl.BlockSpec((B, W), lambda i: (i, 0)),
            pl.BlockSpec((B, D), lambda i: (i, 0)),
        ],
        out_shape=[
            jax.ShapeDtypeStruct((N_NODES, W), jnp.float32),
            jax.ShapeDtypeStruct((N_NODES, W), jnp.float32),
            jax.ShapeDtypeStruct((N_NODES, W), jnp.float32),
            jax.ShapeDtypeStruct((N_NODES, W), jnp.float32),
            jax.ShapeDtypeStruct((N_NODES, D), jnp.float32),
        ],
    )(x, W1, b1r, Wsl)


def _sc_body(src_hbm, dst_hbm, erel_hbm, xg0_hbm, xg1_hbm, xg2_hbm, xg3_hbm,
             outE_hbm, outW_hbm, outP_hbm,
             accE, accW, accP,
             srcb0, srcb1, srcb2, dstb0, dstb1, dstb2,
             erelb0, erelb1, erelb2, w1b0, w1b1, w1b2,
             prodb0, prodb1, prodb2,
             ses0, ses1, ses2, sed0, sed1, sed2,
             see0, see1, see2, seg0, seg1, seg2,
             sesc0, sesc1, sesc2):
    c = lax.axis_index("c")
    s = lax.axis_index("s")
    srcb = [srcb0, srcb1, srcb2]
    dstb = [dstb0, dstb1, dstb2]
    erelb = [erelb0, erelb1, erelb2]
    w1b = [w1b0, w1b1, w1b2]
    prodb = [prodb0, prodb1, prodb2]
    ses = [ses0, ses1, ses2]
    sed = [sed0, sed1, sed2]
    see = [see0, see1, see2]
    seg = [seg0, seg1, seg2]
    sesc = [sesc0, sesc1, sesc2]

    zeros16 = jnp.zeros((16,), jnp.float32)
    r0 = s * ROWS_PER_SUB
    e_base = s * EPS

    def zero_strip():
        def _zb_zero(i, _):
            erelb0[i, pl.ds(0, 16)] = zeros16
            erelb0[i, pl.ds(16, 16)] = zeros16
            return _
        lax.fori_loop(0, CH, _zb_zero, ())
        left = ROWS_PER_SUB
        off = 0
        while left > 0:
            nb = min(CH, left)
            pltpu.sync_copy(erelb0.at[pl.ds(0, nb)], accE.at[pl.ds(r0 + off, nb)])
            pltpu.sync_copy(erelb0.at[pl.ds(0, nb)], accW.at[pl.ds(r0 + off, nb)])
            pltpu.sync_copy(erelb0.at[pl.ds(0, nb)], accP.at[pl.ds(r0 + off, nb)])
            off += nb
            left -= nb

    def phase(col0, xw1_hbm, out_base):
        zero_strip()
        plsc.subcore_barrier()

        def fire_inputs(k, b):
            e0 = e_base + k * CH
            pltpu.async_copy(src_hbm.at[pl.ds(e0, CH)], srcb[b], ses[b])
            pltpu.async_copy(dst_hbm.at[pl.ds(e0, CH)], dstb[b], sed[b])
            pltpu.async_copy(erel_hbm.at[pl.ds(e0, CH), pl.ds(col0, W)],
                             erelb[b], see[b])

        def wait_src(k, b):
            e0 = e_base + k * CH
            pltpu.make_async_copy(src_hbm.at[pl.ds(e0, CH)], srcb[b],
                                  ses[b]).wait()

        def fire_gather(b):
            pltpu.async_copy(xw1_hbm.at[srcb[b]], w1b[b], seg[b])

        def drain_scatters(b):
            pltpu.make_async_copy(erelb[b], accE.at[dstb[b]], sesc[b]).wait()
            pltpu.make_async_copy(w1b[b], accW.at[dstb[b]], sesc[b]).wait()
            pltpu.make_async_copy(prodb[b], accP.at[dstb[b]], sesc[b]).wait()

        def chunk_step(k, b, drain, fire2, gnext):
            e0 = e_base + k * CH
            pltpu.make_async_copy(xw1_hbm.at[srcb[b]], w1b[b], seg[b]).wait()
            pltpu.make_async_copy(erel_hbm.at[pl.ds(e0, CH), pl.ds(col0, W)],
                                  erelb[b], see[b]).wait()

            def _mul(i, _i):
                for r in range(2):
                    for j in range(0, W, 16):
                        prodb[b][2 * i + r, pl.ds(j, 16)] = (
                            erelb[b][2 * i + r, pl.ds(j, 16)]
                            * w1b[b][2 * i + r, pl.ds(j, 16)])
                return _i
            lax.fori_loop(0, CH // 2, _mul, ())

            pltpu.make_async_copy(dst_hbm.at[pl.ds(e0, CH)], dstb[b],
                                  sed[b]).wait()
            pltpu.async_copy(erelb[b], accE.at[dstb[b]], sesc[b], add=True)
            pltpu.async_copy(w1b[b], accW.at[dstb[b]], sesc[b], add=True)
            pltpu.async_copy(prodb[b], accP.at[dstb[b]], sesc[b], add=True)
            dsl = (b + 2) % NSLOT
            if drain:
                drain_scatters(dsl)
            if fire2:
                fire_inputs(k + 2, dsl)
            if gnext:
                nb = (b + 1) % NSLOT
                wait_src(k + 1, nb)
                fire_gather(nb)

        fire_inputs(0, 0)
        fire_inputs(1, 1)
        wait_src(0, 0)
        fire_gather(0)
        chunk_step(0, 0, False, True, True)
        chunk_step(1, 1, True, True, True)

        def _ring_body(g, _):
            for j in range(NSLOT):
                chunk_step(2 + NSLOT * g + j, (2 + j) % NSLOT, True, True, True)
            return _
        lax.fori_loop(0, NGROUPS, _ring_body, ())

        for k in range(2 + NSLOT * NGROUPS, NCH):
            chunk_step(k, k % NSLOT, True, k + 2 < NCH, k + 1 < NCH)
        drain_scatters((NCH - 1) % NSLOT)

        plsc.subcore_barrier()
        out_r0 = out_base + r0
        pltpu.sync_copy(accE.at[pl.ds(r0, ROWS_PER_SUB)],
                        outE_hbm.at[pl.ds(out_r0, ROWS_PER_SUB)])
        pltpu.sync_copy(accW.at[pl.ds(r0, ROWS_PER_SUB)],
                        outW_hbm.at[pl.ds(out_r0, ROWS_PER_SUB)])
        pltpu.sync_copy(accP.at[pl.ds(r0, ROWS_PER_SUB)],
                        outP_hbm.at[pl.ds(out_r0, ROWS_PER_SUB)])

    def run_core(xw1s, gbase):
        for p in range(2):
            g = gbase + p
            phase(g * W, xw1s[p], g * NPAD)

    pl.when(c == 0)(lambda: run_core([xg0_hbm, xg1_hbm], 0))
    pl.when(c == 1)(lambda: run_core([xg2_hbm, xg3_hbm], 2))


def _sc(src, dst, erel, xg0, xg1, xg2, xg3):
    mesh = plsc.VectorSubcoreMesh(core_axis_name="c", subcore_axis_name="s")
    f = pl.kernel(
        _sc_body,
        out_type=[
            jax.ShapeDtypeStruct((NG * NPAD, W), jnp.float32),
            jax.ShapeDtypeStruct((NG * NPAD, W), jnp.float32),
            jax.ShapeDtypeStruct((NG * NPAD, W), jnp.float32),
        ],
        mesh=mesh,
        compiler_params=_SC_PARAMS,
        scratch_types=(
            [pltpu.VMEM_SHARED((NPAD, W), jnp.float32)] * 3
            + [pltpu.VMEM((CH,), jnp.int32)] * 6
            + [pltpu.VMEM((CH, W), jnp.float32)] * 9
            + [pltpu.SemaphoreType.DMA] * 15
        ),
    )
    return f(src, dst, erel, xg0, xg1, xg2, xg3)


def _deg_body(dst_hbm, dego_hbm, degl, dstb):
    c = lax.axis_index("c")
    s = lax.axis_index("s")
    w = s * 2 + c

    zeros16 = jnp.zeros((16,), jnp.float32)
    ones16 = jnp.ones((16,), jnp.float32)

    def _deg_zero(t, _):
        degl[pl.ds(t * 16, 16)] = zeros16
        return _
    lax.fori_loop(0, NPAD // 16, _deg_zero, ())

    e_base = w * DEPS

    def _chunk_body(k, _):
        pltpu.sync_copy(dst_hbm.at[pl.ds(e_base + k * DCH, DCH)], dstb)

        def _acc(j, _i):
            idx16 = dstb[pl.ds(j * 16, 16)]
            plsc.addupdate_scatter(degl, [idx16], ones16)
            return _i
        lax.fori_loop(0, DCH // 16, _acc, ())
        return _
    lax.fori_loop(0, DNCH, _chunk_body, ())

    pltpu.sync_copy(degl, dego_hbm.at[pl.ds(w * NPAD, NPAD)])


def _scdeg(dst):
    mesh = plsc.VectorSubcoreMesh(core_axis_name="c", subcore_axis_name="s")
    f = pl.kernel(
        _deg_body,
        out_type=jax.ShapeDtypeStruct((NW * NPAD,), jnp.float32),
        mesh=mesh,
        compiler_params=_SC_PARAMS,
        scratch_types=[
            pltpu.VMEM((NPAD,), jnp.float32),
            pltpu.VMEM((DCH,), jnp.int32),
        ],
    )
    return f(dst)


def _post_body(e4_ref, w4_ref, p4_ref, dgp_ref, xsl_ref,
               w2_ref, b2_ref, wline_ref, bline_ref, out_ref):
    Se = jnp.concatenate([e4_ref[0], e4_ref[1], e4_ref[2], e4_ref[3]], axis=1)
    Sw = jnp.concatenate([w4_ref[0], w4_ref[1], w4_ref[2], w4_ref[3]], axis=1)
    Sp = jnp.concatenate([p4_ref[0], p4_ref[1], p4_ref[2], p4_ref[3]], axis=1)
    deg = jnp.sum(dgp_ref[...], axis=1)
    w2 = w2_ref[...]
    W2a = w2[0:D]
    W2b = w2[D:2 * D]
    W2c = w2[2 * D:3 * D]
    nei = (jnp.dot(Se, W2a + W2b, preferred_element_type=jnp.float32)
           + jnp.dot(Sw, W2a - W2b, preferred_element_type=jnp.float32)
           + jnp.dot(Sp, W2c, preferred_element_type=jnp.float32)
           + deg[:, None] * b2_ref[...])
    has_in = (deg > 0).astype(jnp.float32)[:, None]
    node = nei + xsl_ref[...] * has_in
    out = jnp.dot(node, wline_ref[...], preferred_element_type=jnp.float32) + bline_ref[...]
    out_ref[...] = jnp.maximum(out, 0.0)


def _post(e4, w4, p4, dgp, xsl, W2, b2r, Wline, bliner):
    B = 2000
    g = N_NODES // B
    return pl.pallas_call(
        _post_body,
        grid=(g,),
        in_specs=[
            pl.BlockSpec((NG, B, W), lambda i: (0, i, 0)),
            pl.BlockSpec((NG, B, W), lambda i: (0, i, 0)),
            pl.BlockSpec((NG, B, W), lambda i: (0, i, 0)),
            pl.BlockSpec((B, NW), lambda i: (i, 0)),
            pl.BlockSpec((B, D), lambda i: (i, 0)),
            pl.BlockSpec((3 * D, D), lambda i: (0, 0)),
            pl.BlockSpec((1, D), lambda i: (0, 0)),
            pl.BlockSpec((D, D), lambda i: (0, 0)),
            pl.BlockSpec((1, D), lambda i: (0, 0)),
        ],
        out_specs=pl.BlockSpec((B, D), lambda i: (i, 0)),
        out_shape=jax.ShapeDtypeStruct((N_NODES, D), jnp.float32),
    )(e4, w4, p4, dgp, xsl, W2, b2r, Wline, bliner)


def kernel(x, edge_index, norm, edge_rel_emd, target_rel_emd_new,
           W1, b1, W2, b2, Wsl, Wline, bline):
    del norm, target_rel_emd_new
    src = edge_index[0]
    dst = edge_index[1]
    xg0, xg1, xg2, xg3, xsl = _pre(x, W1, b1.reshape(1, D), Wsl)
    outE, outW, outP = _sc(src, dst, edge_rel_emd, xg0, xg1, xg2, xg3)
    degp = _scdeg(dst)
    e4 = outE.reshape(NG, NPAD, W)
    w4 = outW.reshape(NG, NPAD, W)
    p4 = outP.reshape(NG, NPAD, W)
    dgp = degp.reshape(NW, NPAD).T
    return _post(e4, w4, p4, dgp, xsl, W2, b2.reshape(1, D),
                 Wline, bline.reshape(1, D))
